# Initial kernel scaffold; baseline (speedup 1.0000x reference)
import jax
import jax.numpy as jnp
from jax.experimental import pallas as pl


def kernel(node_features, edge_features, senders, receivers, params):
    raise NotImplementedError("write your pallas kernel here")


# speedup vs baseline: 2.3643x; 2.3643x over previous
import functools

import jax
import jax.numpy as jnp
from jax import lax
from jax.experimental import pallas as pl
from jax.experimental.pallas import tpu as pltpu
from jax.experimental.pallas import tpu_sc as plsc

_N = 10000
_E = 320000
_LAT = 128
_OUT = 3
_EPS = 1e-5

_RE = 1280
_RN = 2000
_NC, _NS = 2, 16
_NW = _NC * _NS
_EPW = _E // _NW
_C = 80
_NPAD = 10240
_RPT = _NPAD // _NS
_ZR = 128


def _ln(y, g, b):
    mu = jnp.mean(y, axis=-1, keepdims=True)
    var = jnp.mean(y * y, axis=-1, keepdims=True) - mu * mu
    return (y - mu) * lax.rsqrt(var + _EPS) * g + b


def _dot(x, w):
    return jnp.dot(x, w, preferred_element_type=jnp.float32)


def _node_enc_body(x, w1, b1, w2, b2, w3, b3, g, bb, ws, wr, b1e,
                   o_lat, o_sw, o_rw):
    h = jnp.maximum(_dot(x[...], w1[...]) + b1[...], 0.0)
    h = jnp.maximum(_dot(h, w2[...]) + b2[...], 0.0)
    y = _ln(_dot(h, w3[...]) + b3[...], g[...], bb[...])
    o_lat[...] = y
    o_sw[...] = _dot(y, ws[...]) + b1e[...]
    o_rw[...] = _dot(y, wr[...])


def _edge_enc_body(xt, w1, b1, w2, b2, w3, b3, g, bb, o_lat):
    h = lax.dot_general(xt[...], w1[...], (((0,), (0,)), ((), ())),
                        preferred_element_type=jnp.float32)
    h = jnp.maximum(h + b1[...], 0.0)
    h = jnp.maximum(_dot(h, w2[...]) + b2[...], 0.0)
    o_lat[...] = _ln(_dot(h, w3[...]) + b3[...], g[...], bb[...])


def _edge_step_body(g_in, elat, we, w2, b2, w3, b3, lg, lb, o_ne, o_eout):
    e = elat[...]
    h = jnp.maximum(g_in[...] + _dot(e, we[...]), 0.0)
    h = jnp.maximum(_dot(h, w2[...]) + b2[...], 0.0)
    ne = _ln(_dot(h, w3[...]) + b3[...], lg[...], lb[...])
    o_ne[...] = ne
    o_eout[...] = e + ne


def _node_step_body(nlat, parts, wn, wa, b1, w2, b2, w3, b3, g, bb,
                    ws, wr, b1e, o_lat, o_sw, o_rw):
    n = nlat[...]
    agg = parts[0] + parts[1]
    h = jnp.maximum(_dot(n, wn[...]) + _dot(agg, wa[...]) + b1[...], 0.0)
    h = jnp.maximum(_dot(h, w2[...]) + b2[...], 0.0)
    y = _ln(_dot(h, w3[...]) + b3[...], g[...], bb[...])
    no = n + y
    o_lat[...] = no
    o_sw[...] = _dot(no, ws[...]) + b1e[...]
    o_rw[...] = _dot(no, wr[...])


def _node_final_body(nlat, parts, wn, wa, b1, w2, b2, w3, b3, g, bb,
                     d1, e1, d2, e2, d3, e3, o_dec):
    n = nlat[...]
    agg = parts[0] + parts[1]
    h = jnp.maximum(_dot(n, wn[...]) + _dot(agg, wa[...]) + b1[...], 0.0)
    h = jnp.maximum(_dot(h, w2[...]) + b2[...], 0.0)
    y = _ln(_dot(h, w3[...]) + b3[...], g[...], bb[...])
    no = n + y
    h = jnp.maximum(_dot(no, d1[...]) + e1[...], 0.0)
    h = jnp.maximum(_dot(h, d2[...]) + e2[...], 0.0)
    o_dec[...] = _dot(h, d3[...]) + e3[...]


def _full_spec(a):
    nd = a.ndim
    return pl.BlockSpec(a.shape, lambda i, _nd=nd: (0,) * _nd)


def _row_spec(rows, cols):
    return pl.BlockSpec((rows, cols), lambda i: (i, 0))


def _tc_call(body, row_args, weight_args, out_shapes, rows, total_rows):
    grid = (total_rows // rows,)
    in_specs = [
        (pl.BlockSpec((_NC, rows, a.shape[-1]), lambda i: (0, i, 0))
         if a.ndim == 3 else _row_spec(rows, a.shape[-1]))
        for a in row_args
    ]
    in_specs += [_full_spec(a) for a in weight_args]
    out_specs = [_row_spec(rows, s[-1]) for s in out_shapes]
    outs = pl.pallas_call(
        body,
        grid=grid,
        in_specs=in_specs,
        out_specs=out_specs if len(out_specs) > 1 else out_specs[0],
        out_shape=([jax.ShapeDtypeStruct(s, jnp.float32) for s in out_shapes]
                   if len(out_shapes) > 1
                   else jax.ShapeDtypeStruct(out_shapes[0], jnp.float32)),
        compiler_params=pltpu.CompilerParams(
            dimension_semantics=("parallel",)),
    )(*row_args, *weight_args)
    return outs


@functools.cache
def _sc_mesh():
    return plsc.VectorSubcoreMesh(core_axis_name="c", subcore_axis_name="s",
                                  num_cores=_NC, num_subcores=_NS)


@functools.cache
def _gather_add_kernel():
    @functools.partial(
        pl.kernel,
        out_type=jax.ShapeDtypeStruct((_E, _LAT), jnp.float32),
        mesh=_sc_mesh(),
        scratch_types=[
            pltpu.VMEM((_C,), jnp.int32),
            pltpu.VMEM((_C,), jnp.int32),
            pltpu.VMEM((_C, _LAT), jnp.float32),
            pltpu.VMEM((_C, _LAT), jnp.float32),
            pltpu.SemaphoreType.DMA,
            pltpu.SemaphoreType.DMA,
        ],
    )
    def k(sw_hbm, rw_hbm, snd_hbm, rcv_hbm, out_hbm,
          idx_s, idx_r, buf_s, buf_r, sem_s, sem_r):
        c = lax.axis_index("c")
        s = lax.axis_index("s")
        base = (c * _NS + s) * _EPW

        def chunk(i, carry):
            row0 = pl.multiple_of(base + i * _C, 8)
            pltpu.sync_copy(snd_hbm.at[pl.ds(row0, _C)], idx_s)
            pltpu.sync_copy(rcv_hbm.at[pl.ds(row0, _C)], idx_r)
            cp_s = pltpu.async_copy(sw_hbm.at[idx_s], buf_s, sem_s)
            cp_r = pltpu.async_copy(rw_hbm.at[idx_r], buf_r, sem_r)
            cp_s.wait()
            cp_r.wait()

            def addrow(r, carry2):
                for j in range(_LAT // 16):
                    sl = pl.ds(j * 16, 16)
                    buf_s[r, sl] = buf_s[r, sl] + buf_r[r, sl]
                return carry2

            lax.fori_loop(0, _C, addrow, 0, unroll=2)
            pltpu.sync_copy(buf_s, out_hbm.at[pl.ds(row0, _C)])
            return carry

        lax.fori_loop(0, _EPW // _C, chunk, 0)

    return k


@functools.cache
def _scatter_add_kernel():
    @functools.partial(
        pl.kernel,
        out_type=jax.ShapeDtypeStruct((_NC, _NPAD, _LAT), jnp.float32),
        mesh=_sc_mesh(),
        scratch_types=[
            pltpu.VMEM((_C,), jnp.int32),
            pltpu.VMEM((_C, _LAT), jnp.float32),
            pltpu.VMEM((_ZR, _LAT), jnp.float32),
            pltpu.VMEM_SHARED((_NPAD, _LAT), jnp.float32),
        ],
    )
    def k(ne_hbm, rcv_hbm, out_hbm, idx, buf, zbuf, acc):
        c = lax.axis_index("c")
        s = lax.axis_index("s")
        zero16 = jnp.zeros((16,), jnp.float32)

        def zrow(r, carry):
            for j in range(_LAT // 16):
                zbuf[r, pl.ds(j * 16, 16)] = zero16
            return carry

        lax.fori_loop(0, _ZR, zrow, 0)
        my_r0 = s * _RPT

        def zchunk(i, carry):
            pltpu.sync_copy(zbuf, acc.at[pl.ds(my_r0 + i * _ZR, _ZR)])
            return carry

        lax.fori_loop(0, _RPT // _ZR, zchunk, 0)
        plsc.subcore_barrier()

        base = (c * _NS + s) * _EPW

        def chunk(i, carry):
            row0 = pl.multiple_of(base + i * _C, 8)
            pltpu.sync_copy(rcv_hbm.at[pl.ds(row0, _C)], idx)
            pltpu.sync_copy(ne_hbm.at[pl.ds(row0, _C)], buf)
            pltpu.sync_copy(buf, acc.at[idx], add=True)
            return carry

        lax.fori_loop(0, _EPW // _C, chunk, 0)
        plsc.subcore_barrier()

        def wchunk(i, carry):
            r0 = my_r0 + i * _ZR
            pltpu.sync_copy(acc.at[pl.ds(r0, _ZR)], out_hbm.at[c, pl.ds(r0, _ZR)])
            return carry

        lax.fori_loop(0, _RPT // _ZR, wchunk, 0)

    return k


def _sc_gather_add(sw, rw, senders, receivers):
    return _gather_add_kernel()(sw, rw, senders, receivers)


def _sc_scatter_partials(ne, receivers):
    return _scatter_add_kernel()(ne, receivers)


def _r(b):
    return b.reshape(1, -1)


def kernel(node_features, edge_features, senders, receivers, params):
    p = params
    (nw1, nb1), (nw2, nb2), (nw3, nb3) = p['node_enc']['mlp']
    ng, nb = p['node_enc']['ln']
    (ew1, eb1), (ew2, eb2), (ew3, eb3) = p['edge_enc']['mlp']
    eg, eb = p['edge_enc']['ln']
    (d1, f1), (d2, f2), (d3, f3) = p['dec']

    blocks = []
    for bp in p['blocks']:
        (w1e, b1e), (w2e, b2e), (w3e, b3e) = bp['edge']['mlp']
        (w1n, b1n), (w2n, b2n), (w3n, b3n) = bp['node']['mlp']
        blocks.append(dict(
            ws=w1e[:_LAT], wr=w1e[_LAT:2 * _LAT], we=w1e[2 * _LAT:],
            b1e=_r(b1e), w2e=w2e, b2e=_r(b2e), w3e=w3e, b3e=_r(b3e),
            ge=_r(bp['edge']['ln'][0]), be=_r(bp['edge']['ln'][1]),
            wn=w1n[:_LAT], wa=w1n[_LAT:], b1n=_r(b1n),
            w2n=w2n, b2n=_r(b2n), w3n=w3n, b3n=_r(b3n),
            gn=_r(bp['node']['ln'][0]), bn=_r(bp['node']['ln'][1]),
        ))

    d3p = jnp.zeros((_LAT, _LAT), jnp.float32).at[:, :_OUT].set(d3)
    f3p = jnp.zeros((1, _LAT), jnp.float32).at[:, :_OUT].set(_r(f3))

    node_lat, sw, rw = _tc_call(
        _node_enc_body, [node_features],
        [nw1, _r(nb1), nw2, _r(nb2), nw3, _r(nb3), _r(ng), _r(nb),
         blocks[0]['ws'], blocks[0]['wr'], blocks[0]['b1e']],
        [(_N, _LAT)] * 3, _RN, _N)

    eft = edge_features.T
    ew_args = [ew1, _r(eb1), ew2, _r(eb2), ew3, _r(eb3), _r(eg), _r(eb)]
    edge_lat = pl.pallas_call(
        _edge_enc_body,
        grid=(_E // _RE,),
        in_specs=([pl.BlockSpec((4, _RE), lambda i: (0, i))]
                  + [_full_spec(a) for a in ew_args]),
        out_specs=_row_spec(_RE, _LAT),
        out_shape=jax.ShapeDtypeStruct((_E, _LAT), jnp.float32),
        compiler_params=pltpu.CompilerParams(
            dimension_semantics=("parallel",)),
    )(eft, *ew_args)

    for s, bl in enumerate(blocks):
        g = _sc_gather_add(sw, rw, senders, receivers)
        ne, edge_lat = _tc_call(
            _edge_step_body, [g, edge_lat],
            [bl['we'], bl['w2e'], bl['b2e'], bl['w3e'], bl['b3e'],
             bl['ge'], bl['be']],
            [(_E, _LAT)] * 2, _RE, _E)
        parts = _sc_scatter_partials(ne, receivers)
        if s < len(blocks) - 1:
            nxt = blocks[s + 1]
            node_lat, sw, rw = _tc_call(
                _node_step_body, [node_lat, parts],
                [bl['wn'], bl['wa'], bl['b1n'], bl['w2n'], bl['b2n'],
                 bl['w3n'], bl['b3n'], bl['gn'], bl['bn'],
                 nxt['ws'], nxt['wr'], nxt['b1e']],
                [(_N, _LAT)] * 3, _RN, _N)
        else:
            dec = _tc_call(
                _node_final_body, [node_lat, parts],
                [bl['wn'], bl['wa'], bl['b1n'], bl['w2n'], bl['b2n'],
                 bl['w3n'], bl['b3n'], bl['gn'], bl['bn'],
                 d1, _r(f1), d2, _r(f2), d3p, f3p],
                [(_N, _LAT)], _RN, _N)
    return dec[:, :_OUT]


# speedup vs baseline: 3.4011x; 1.4385x over previous
import functools

import jax
import jax.numpy as jnp
from jax import lax
from jax.experimental import pallas as pl
from jax.experimental.pallas import tpu as pltpu
from jax.experimental.pallas import tpu_sc as plsc

_N = 10000
_E = 320000
_LAT = 128
_OUT = 3
_EPS = 1e-5

_RE = 1280
_RN = 2000
_NC, _NS = 2, 16
_NW = _NC * _NS
_EPW = _E // _NW
_C = 80
_NPAD = 10240
_RPT = _NPAD // _NS
_ZR = 128


def _ln(y, g, b):
    mu = jnp.mean(y, axis=-1, keepdims=True)
    yc = y - mu
    var = jnp.mean(yc * yc, axis=-1, keepdims=True)
    return yc / jnp.sqrt(var + _EPS) * g + b


def _dot(x, w):
    return jnp.dot(x, w, preferred_element_type=jnp.float32)


def _node_enc_body(x, w1, b1, w2, b2, w3, b3, g, bb, ws, wr, b1e,
                   o_lat, o_sw, o_rw):
    h = jnp.maximum(_dot(x[...], w1[...]) + b1[...], 0.0)
    h = jnp.maximum(_dot(h, w2[...]) + b2[...], 0.0)
    y = _ln(_dot(h, w3[...]) + b3[...], g[...], bb[...])
    o_lat[...] = y
    o_sw[...] = _dot(y, ws[...]) + b1e[...]
    o_rw[...] = _dot(y, wr[...])


def _edge_enc_body(xt, w1, b1, w2, b2, w3, b3, g, bb, o_lat):
    h = lax.dot_general(xt[...], w1[...], (((0,), (0,)), ((), ())),
                        preferred_element_type=jnp.float32)
    h = jnp.maximum(h + b1[...], 0.0)
    h = jnp.maximum(_dot(h, w2[...]) + b2[...], 0.0)
    o_lat[...] = _ln(_dot(h, w3[...]) + b3[...], g[...], bb[...])


def _edge_step_body(g_in, elat, we, w2, b2, w3, b3, lg, lb, o_ne, o_eout):
    e = elat[...]
    h = jnp.maximum(g_in[...] + _dot(e, we[...]), 0.0)
    h = jnp.maximum(_dot(h, w2[...]) + b2[...], 0.0)
    ne = _ln(_dot(h, w3[...]) + b3[...], lg[...], lb[...])
    o_ne[...] = ne
    o_eout[...] = e + ne


def _node_step_body(nlat, parts, wn, wa, b1, w2, b2, w3, b3, g, bb,
                    ws, wr, b1e, o_lat, o_sw, o_rw):
    n = nlat[...]
    agg = parts[0] + parts[1]
    h = jnp.maximum(_dot(n, wn[...]) + _dot(agg, wa[...]) + b1[...], 0.0)
    h = jnp.maximum(_dot(h, w2[...]) + b2[...], 0.0)
    y = _ln(_dot(h, w3[...]) + b3[...], g[...], bb[...])
    no = n + y
    o_lat[...] = no
    o_sw[...] = _dot(no, ws[...]) + b1e[...]
    o_rw[...] = _dot(no, wr[...])


def _node_final_body(nlat, parts, wn, wa, b1, w2, b2, w3, b3, g, bb,
                     d1, e1, d2, e2, d3, e3, o_dec):
    n = nlat[...]
    agg = parts[0] + parts[1]
    h = jnp.maximum(_dot(n, wn[...]) + _dot(agg, wa[...]) + b1[...], 0.0)
    h = jnp.maximum(_dot(h, w2[...]) + b2[...], 0.0)
    y = _ln(_dot(h, w3[...]) + b3[...], g[...], bb[...])
    no = n + y
    h = jnp.maximum(_dot(no, d1[...]) + e1[...], 0.0)
    h = jnp.maximum(_dot(h, d2[...]) + e2[...], 0.0)
    o_dec[...] = _dot(h, d3[...]) + e3[...]


def _full_spec(a):
    nd = a.ndim
    return pl.BlockSpec(a.shape, lambda i, _nd=nd: (0,) * _nd)


def _row_spec(rows, cols):
    return pl.BlockSpec((rows, cols), lambda i: (i, 0))


def _tc_call(body, row_args, weight_args, out_shapes, rows, total_rows):
    grid = (total_rows // rows,)
    in_specs = [
        (pl.BlockSpec((_NC, rows, a.shape[-1]), lambda i: (0, i, 0))
         if a.ndim == 3 else _row_spec(rows, a.shape[-1]))
        for a in row_args
    ]
    in_specs += [_full_spec(a) for a in weight_args]
    out_specs = [_row_spec(rows, s[-1]) for s in out_shapes]
    outs = pl.pallas_call(
        body,
        grid=grid,
        in_specs=in_specs,
        out_specs=out_specs if len(out_specs) > 1 else out_specs[0],
        out_shape=([jax.ShapeDtypeStruct(s, jnp.float32) for s in out_shapes]
                   if len(out_shapes) > 1
                   else jax.ShapeDtypeStruct(out_shapes[0], jnp.float32)),
        compiler_params=pltpu.CompilerParams(
            dimension_semantics=("parallel",)),
    )(*row_args, *weight_args)
    return outs


@functools.cache
def _sc_mesh():
    return plsc.VectorSubcoreMesh(core_axis_name="c", subcore_axis_name="s",
                                  num_cores=_NC, num_subcores=_NS)


_NCH = _EPW // _C


@functools.cache
def _gather_add_kernel():
    @functools.partial(
        pl.kernel,
        out_type=jax.ShapeDtypeStruct((_E, _LAT), jnp.float32),
        mesh=_sc_mesh(),
        scratch_types=[
            pltpu.VMEM((_NCH, _C), jnp.int32),
            pltpu.VMEM((_NCH, _C), jnp.int32),
            pltpu.VMEM((_C, _LAT), jnp.float32),
            pltpu.VMEM((_C, _LAT), jnp.float32),
            pltpu.VMEM((_C, _LAT), jnp.float32),
            pltpu.VMEM((_C, _LAT), jnp.float32),
            pltpu.SemaphoreType.DMA,
            pltpu.SemaphoreType.DMA,
            pltpu.SemaphoreType.DMA,
            pltpu.SemaphoreType.DMA,
        ],
    )
    def k(sw_hbm, rw_hbm, snd_hbm, rcv_hbm, out_hbm,
          idx_s, idx_r, bs0, br0, bs1, br1, g0, g1, w0, w1):
        c = lax.axis_index("c")
        s = lax.axis_index("s")
        wid = c * _NS + s
        base = wid * _EPW

        pltpu.sync_copy(snd_hbm.at[wid], idx_s)
        pltpu.sync_copy(rcv_hbm.at[wid], idx_r)

        def issue(j, bs, br, sem):
            pltpu.async_copy(sw_hbm.at[idx_s.at[j]], bs, sem)
            pltpu.async_copy(rw_hbm.at[idx_r.at[j]], br, sem)

        def drain_g(bs, br, sem):
            pltpu.make_async_copy(sw_hbm.at[idx_s.at[0]], bs, sem).wait()
            pltpu.make_async_copy(rw_hbm.at[idx_r.at[0]], br, sem).wait()

        def add_write(j, bs, br, wsem):
            def addrow(r, carry2):
                for q in range(_LAT // 16):
                    sl = pl.ds(q * 16, 16)
                    bs[r, sl] = bs[r, sl] + br[r, sl]
                return carry2

            lax.fori_loop(0, _C, addrow, 0, unroll=2)
            row0 = pl.multiple_of(base + j * _C, 8)
            pltpu.async_copy(bs, out_hbm.at[pl.ds(row0, _C)], wsem)

        def drain_w(bs, wsem):
            pltpu.make_async_copy(bs, out_hbm.at[pl.ds(base, _C)], wsem).wait()

        issue(0, bs0, br0, g0)

        def body(i, carry):
            j1 = 2 * i + 1

            @pl.when(i > 0)
            def _():
                drain_w(bs1, w1)

            issue(j1, bs1, br1, g1)
            drain_g(bs0, br0, g0)
            add_write(2 * i, bs0, br0, w0)
            drain_w(bs0, w0)
            issue(j1 + 1, bs0, br0, g0)
            drain_g(bs1, br1, g1)
            add_write(j1, bs1, br1, w1)
            return carry

        lax.fori_loop(0, (_NCH - 1) // 2, body, 0)
        drain_w(bs1, w1)
        drain_g(bs0, br0, g0)
        add_write(_NCH - 1, bs0, br0, w0)
        drain_w(bs0, w0)

    return k


@functools.cache
def _scatter_add_kernel():
    @functools.partial(
        pl.kernel,
        out_type=jax.ShapeDtypeStruct((_NC, _NPAD, _LAT), jnp.float32),
        mesh=_sc_mesh(),
        scratch_types=[
            pltpu.VMEM((_NCH, _C), jnp.int32),
            pltpu.VMEM((_C, _LAT), jnp.float32),
            pltpu.VMEM((_C, _LAT), jnp.float32),
            pltpu.VMEM_SHARED((_NPAD, _LAT), jnp.float32),
            pltpu.SemaphoreType.DMA,
            pltpu.SemaphoreType.DMA,
        ],
    )
    def k(ne_hbm, rcv_hbm, out_hbm, idx, b0, b1, acc, r0sem, r1sem):
        c = lax.axis_index("c")
        s = lax.axis_index("s")
        wid = c * _NS + s
        base = wid * _EPW
        zero16 = jnp.zeros((16,), jnp.float32)

        pltpu.sync_copy(rcv_hbm.at[wid], idx)

        def zrow(r, carry):
            for j in range(_LAT // 16):
                b0[r, pl.ds(j * 16, 16)] = zero16
            return carry

        lax.fori_loop(0, _C, zrow, 0)
        my_r0 = s * _RPT

        def zchunk(i, carry):
            pltpu.sync_copy(b0, acc.at[pl.ds(my_r0 + i * _C, _C)])
            return carry

        lax.fori_loop(0, _RPT // _C, zchunk, 0)
        plsc.subcore_barrier()

        def issue(j, b, sem):
            row0 = pl.multiple_of(base + j * _C, 8)
            pltpu.async_copy(ne_hbm.at[pl.ds(row0, _C)], b, sem)

        def drain(b, sem):
            pltpu.make_async_copy(ne_hbm.at[pl.ds(base, _C)], b, sem).wait()

        issue(0, b0, r0sem)

        def body(i, carry):
            j1 = 2 * i + 1
            issue(j1, b1, r1sem)
            drain(b0, r0sem)
            pltpu.sync_copy(b0, acc.at[idx.at[2 * i]], add=True)
            issue(j1 + 1, b0, r0sem)
            drain(b1, r1sem)
            pltpu.sync_copy(b1, acc.at[idx.at[j1]], add=True)
            return carry

        lax.fori_loop(0, (_NCH - 1) // 2, body, 0)
        drain(b0, r0sem)
        pltpu.sync_copy(b0, acc.at[idx.at[_NCH - 1]], add=True)
        plsc.subcore_barrier()

        def wchunk(i, carry):
            r0 = my_r0 + i * _ZR
            pltpu.sync_copy(acc.at[pl.ds(r0, _ZR)], out_hbm.at[c, pl.ds(r0, _ZR)])
            return carry

        lax.fori_loop(0, _RPT // _ZR, wchunk, 0)

    return k


def _sc_gather_add(sw, rw, snd3, rcv3):
    return _gather_add_kernel()(sw, rw, snd3, rcv3)


def _sc_scatter_partials(ne, rcv3):
    return _scatter_add_kernel()(ne, rcv3)


def _r(b):
    return b.reshape(1, -1)


def kernel(node_features, edge_features, senders, receivers, params):
    p = params
    (nw1, nb1), (nw2, nb2), (nw3, nb3) = p['node_enc']['mlp']
    ng, nb = p['node_enc']['ln']
    (ew1, eb1), (ew2, eb2), (ew3, eb3) = p['edge_enc']['mlp']
    eg, eb = p['edge_enc']['ln']
    (d1, f1), (d2, f2), (d3, f3) = p['dec']

    blocks = []
    for bp in p['blocks']:
        (w1e, b1e), (w2e, b2e), (w3e, b3e) = bp['edge']['mlp']
        (w1n, b1n), (w2n, b2n), (w3n, b3n) = bp['node']['mlp']
        blocks.append(dict(
            ws=w1e[:_LAT], wr=w1e[_LAT:2 * _LAT], we=w1e[2 * _LAT:],
            b1e=_r(b1e), w2e=w2e, b2e=_r(b2e), w3e=w3e, b3e=_r(b3e),
            ge=_r(bp['edge']['ln'][0]), be=_r(bp['edge']['ln'][1]),
            wn=w1n[:_LAT], wa=w1n[_LAT:], b1n=_r(b1n),
            w2n=w2n, b2n=_r(b2n), w3n=w3n, b3n=_r(b3n),
            gn=_r(bp['node']['ln'][0]), bn=_r(bp['node']['ln'][1]),
        ))

    d3p = jnp.zeros((_LAT, _LAT), jnp.float32).at[:, :_OUT].set(d3)
    f3p = jnp.zeros((1, _LAT), jnp.float32).at[:, :_OUT].set(_r(f3))

    node_lat, sw, rw = _tc_call(
        _node_enc_body, [node_features],
        [nw1, _r(nb1), nw2, _r(nb2), nw3, _r(nb3), _r(ng), _r(nb),
         blocks[0]['ws'], blocks[0]['wr'], blocks[0]['b1e']],
        [(_N, _LAT)] * 3, _RN, _N)

    eft = edge_features.T
    ew_args = [ew1, _r(eb1), ew2, _r(eb2), ew3, _r(eb3), _r(eg), _r(eb)]
    edge_lat = pl.pallas_call(
        _edge_enc_body,
        grid=(_E // _RE,),
        in_specs=([pl.BlockSpec((4, _RE), lambda i: (0, i))]
                  + [_full_spec(a) for a in ew_args]),
        out_specs=_row_spec(_RE, _LAT),
        out_shape=jax.ShapeDtypeStruct((_E, _LAT), jnp.float32),
        compiler_params=pltpu.CompilerParams(
            dimension_semantics=("parallel",)),
    )(eft, *ew_args)

    snd3 = senders.reshape(_NW, _NCH, _C)
    rcv3 = receivers.reshape(_NW, _NCH, _C)
    for s, bl in enumerate(blocks):
        g = _sc_gather_add(sw, rw, snd3, rcv3)
        ne, edge_lat = _tc_call(
            _edge_step_body, [g, edge_lat],
            [bl['we'], bl['w2e'], bl['b2e'], bl['w3e'], bl['b3e'],
             bl['ge'], bl['be']],
            [(_E, _LAT)] * 2, _RE, _E)
        parts = _sc_scatter_partials(ne, rcv3)
        if s < len(blocks) - 1:
            nxt = blocks[s + 1]
            node_lat, sw, rw = _tc_call(
                _node_step_body, [node_lat, parts],
                [bl['wn'], bl['wa'], bl['b1n'], bl['w2n'], bl['b2n'],
                 bl['w3n'], bl['b3n'], bl['gn'], bl['bn'],
                 nxt['ws'], nxt['wr'], nxt['b1e']],
                [(_N, _LAT)] * 3, _RN, _N)
        else:
            dec = _tc_call(
                _node_final_body, [node_lat, parts],
                [bl['wn'], bl['wa'], bl['b1n'], bl['w2n'], bl['b2n'],
                 bl['w3n'], bl['b3n'], bl['gn'], bl['bn'],
                 d1, _r(f1), d2, _r(f2), d3p, f3p],
                [(_N, _LAT)], _RN, _N)
    return dec[:, :_OUT]


# speedup vs baseline: 3.4770x; 1.0223x over previous
import functools

import jax
import jax.numpy as jnp
from jax import lax
from jax.experimental import pallas as pl
from jax.experimental.pallas import tpu as pltpu
from jax.experimental.pallas import tpu_sc as plsc

_N = 10000
_E = 320000
_LAT = 128
_OUT = 3
_EPS = 1e-5

_RE = 1280
_RN = 2000
_NC, _NS = 2, 16
_NW = _NC * _NS
_EPW = _E // _NW
_C = 80
_NPAD = 10240
_RPT = _NPAD // _NS
_ZR = 128


def _ln(y, g, b):
    mu = jnp.mean(y, axis=-1, keepdims=True)
    yc = y - mu
    var = jnp.mean(yc * yc, axis=-1, keepdims=True)
    return yc / jnp.sqrt(var + _EPS) * g + b


def _dot(x, w):
    return jnp.dot(x, w, preferred_element_type=jnp.float32)


def _node_enc_body(x, w1, b1, w2, b2, w3, b3, g, bb, ws, wr, b1e,
                   o_lat, o_sw, o_rw):
    h = jnp.maximum(_dot(x[...], w1[...]) + b1[...], 0.0)
    h = jnp.maximum(_dot(h, w2[...]) + b2[...], 0.0)
    y = _ln(_dot(h, w3[...]) + b3[...], g[...], bb[...])
    o_lat[...] = y
    o_sw[...] = _dot(y, ws[...]) + b1e[...]
    o_rw[...] = _dot(y, wr[...])


def _edge_enc_body(xt, w1, b1, w2, b2, w3, b3, g, bb, o_lat):
    h = lax.dot_general(xt[...], w1[...], (((0,), (0,)), ((), ())),
                        preferred_element_type=jnp.float32)
    h = jnp.maximum(h + b1[...], 0.0)
    h = jnp.maximum(_dot(h, w2[...]) + b2[...], 0.0)
    o_lat[...] = _ln(_dot(h, w3[...]) + b3[...], g[...], bb[...])


def _edge_step_body(g_in, elat, we, w2, b2, w3, b3, lg, lb, o_ne, o_eout):
    e = elat[...]
    h = jnp.maximum(g_in[...] + _dot(e, we[...]), 0.0)
    h = jnp.maximum(_dot(h, w2[...]) + b2[...], 0.0)
    ne = _ln(_dot(h, w3[...]) + b3[...], lg[...], lb[...])
    o_ne[...] = ne
    o_eout[...] = e + ne


def _edge_last_body(g_in, elat, we, w2, b2, w3, b3, lg, lb, o_ne):
    h = jnp.maximum(g_in[...] + _dot(elat[...], we[...]), 0.0)
    h = jnp.maximum(_dot(h, w2[...]) + b2[...], 0.0)
    o_ne[...] = _ln(_dot(h, w3[...]) + b3[...], lg[...], lb[...])


def _node_step_body(nlat, parts, wn, wa, b1, w2, b2, w3, b3, g, bb,
                    ws, wr, b1e, o_lat, o_sw, o_rw):
    n = nlat[...]
    agg = parts[0] + parts[1]
    h = jnp.maximum(_dot(n, wn[...]) + _dot(agg, wa[...]) + b1[...], 0.0)
    h = jnp.maximum(_dot(h, w2[...]) + b2[...], 0.0)
    y = _ln(_dot(h, w3[...]) + b3[...], g[...], bb[...])
    no = n + y
    o_lat[...] = no
    o_sw[...] = _dot(no, ws[...]) + b1e[...]
    o_rw[...] = _dot(no, wr[...])


def _node_final_body(nlat, parts, wn, wa, b1, w2, b2, w3, b3, g, bb,
                     d1, e1, d2, e2, d3, e3, o_dec):
    n = nlat[...]
    agg = parts[0] + parts[1]
    h = jnp.maximum(_dot(n, wn[...]) + _dot(agg, wa[...]) + b1[...], 0.0)
    h = jnp.maximum(_dot(h, w2[...]) + b2[...], 0.0)
    y = _ln(_dot(h, w3[...]) + b3[...], g[...], bb[...])
    no = n + y
    h = jnp.maximum(_dot(no, d1[...]) + e1[...], 0.0)
    h = jnp.maximum(_dot(h, d2[...]) + e2[...], 0.0)
    o_dec[...] = _dot(h, d3[...]) + e3[...]


def _full_spec(a):
    nd = a.ndim
    return pl.BlockSpec(a.shape, lambda i, _nd=nd: (0,) * _nd)


def _row_spec(rows, cols):
    return pl.BlockSpec((rows, cols), lambda i: (i, 0))


def _tc_call(body, row_args, weight_args, out_shapes, rows, total_rows):
    grid = (total_rows // rows,)
    in_specs = [
        (pl.BlockSpec((_NC, rows, a.shape[-1]), lambda i: (0, i, 0))
         if a.ndim == 3 else _row_spec(rows, a.shape[-1]))
        for a in row_args
    ]
    in_specs += [_full_spec(a) for a in weight_args]
    out_specs = [_row_spec(rows, s[-1]) for s in out_shapes]
    outs = pl.pallas_call(
        body,
        grid=grid,
        in_specs=in_specs,
        out_specs=out_specs if len(out_specs) > 1 else out_specs[0],
        out_shape=([jax.ShapeDtypeStruct(s, jnp.float32) for s in out_shapes]
                   if len(out_shapes) > 1
                   else jax.ShapeDtypeStruct(out_shapes[0], jnp.float32)),
        compiler_params=pltpu.CompilerParams(
            dimension_semantics=("parallel",)),
    )(*row_args, *weight_args)
    return outs


@functools.cache
def _sc_mesh():
    return plsc.VectorSubcoreMesh(core_axis_name="c", subcore_axis_name="s",
                                  num_cores=_NC, num_subcores=_NS)


_NCH = _EPW // _C


@functools.cache
def _gather_add_kernel():
    @functools.partial(
        pl.kernel,
        out_type=jax.ShapeDtypeStruct((_E, _LAT), jnp.float32),
        mesh=_sc_mesh(),
        scratch_types=[
            pltpu.VMEM((_NCH, _C), jnp.int32),
            pltpu.VMEM((_NCH, _C), jnp.int32),
            pltpu.VMEM((_C, _LAT), jnp.float32),
            pltpu.VMEM((_C, _LAT), jnp.float32),
            pltpu.VMEM((_C, _LAT), jnp.float32),
            pltpu.VMEM((_C, _LAT), jnp.float32),
            pltpu.VMEM((_C, _LAT), jnp.float32),
            pltpu.VMEM((_C, _LAT), jnp.float32),
            pltpu.SemaphoreType.DMA,
            pltpu.SemaphoreType.DMA,
            pltpu.SemaphoreType.DMA,
            pltpu.SemaphoreType.DMA,
        ],
    )
    def k(sw_hbm, rw_hbm, snd_hbm, rcv_hbm, out_hbm,
          idx_s, idx_r, bs0, br0, bs1, br1, ob0, ob1, g0, g1, w0, w1):
        c = lax.axis_index("c")
        s = lax.axis_index("s")
        wid = c * _NS + s
        base = wid * _EPW

        pltpu.sync_copy(snd_hbm.at[wid], idx_s)
        pltpu.sync_copy(rcv_hbm.at[wid], idx_r)

        def issue(j, bs, br, sem):
            pltpu.async_copy(sw_hbm.at[idx_s.at[j]], bs, sem)
            pltpu.async_copy(rw_hbm.at[idx_r.at[j]], br, sem)

        def drain_g(bs, br, sem):
            pltpu.make_async_copy(sw_hbm.at[idx_s.at[0]], bs, sem).wait()
            pltpu.make_async_copy(rw_hbm.at[idx_r.at[0]], br, sem).wait()

        def add(bs, br, ob):
            def addrow(r, carry2):
                for q in range(_LAT // 16):
                    sl = pl.ds(q * 16, 16)
                    ob[r, sl] = bs[r, sl] + br[r, sl]
                return carry2

            lax.fori_loop(0, _C, addrow, 0, unroll=2)

        def issue_w(j, ob, wsem):
            row0 = pl.multiple_of(base + j * _C, 8)
            pltpu.async_copy(ob, out_hbm.at[pl.ds(row0, _C)], wsem)

        def drain_w(ob, wsem):
            pltpu.make_async_copy(ob, out_hbm.at[pl.ds(base, _C)], wsem).wait()

        issue(0, bs0, br0, g0)
        issue(1, bs1, br1, g1)
        nb = (_NCH - 1) // 2

        def body(i, carry):
            j0 = 2 * i
            drain_g(bs0, br0, g0)

            @pl.when(i > 0)
            def _():
                drain_w(ob0, w0)

            add(bs0, br0, ob0)

            @pl.when(j0 + 2 < _NCH)
            def _():
                issue(j0 + 2, bs0, br0, g0)

            issue_w(j0, ob0, w0)

            drain_g(bs1, br1, g1)

            @pl.when(i > 0)
            def _():
                drain_w(ob1, w1)

            add(bs1, br1, ob1)

            @pl.when(j0 + 3 < _NCH)
            def _():
                issue(j0 + 3, bs1, br1, g1)

            issue_w(j0 + 1, ob1, w1)
            return carry

        lax.fori_loop(0, nb, body, 0)
        drain_g(bs0, br0, g0)
        drain_w(ob0, w0)
        add(bs0, br0, ob0)
        issue_w(_NCH - 1, ob0, w0)
        drain_w(ob0, w0)
        drain_w(ob1, w1)

    return k


@functools.cache
def _scatter_add_kernel():
    @functools.partial(
        pl.kernel,
        out_type=jax.ShapeDtypeStruct((_NC, _NPAD, _LAT), jnp.float32),
        mesh=_sc_mesh(),
        scratch_types=[
            pltpu.VMEM((_NCH, _C), jnp.int32),
            pltpu.VMEM((_C, _LAT), jnp.float32),
            pltpu.VMEM((_C, _LAT), jnp.float32),
            pltpu.VMEM_SHARED((_NPAD, _LAT), jnp.float32),
            pltpu.SemaphoreType.DMA,
            pltpu.SemaphoreType.DMA,
        ],
    )
    def k(ne_hbm, rcv_hbm, out_hbm, idx, b0, b1, acc, r0sem, r1sem):
        c = lax.axis_index("c")
        s = lax.axis_index("s")
        wid = c * _NS + s
        base = wid * _EPW
        zero16 = jnp.zeros((16,), jnp.float32)

        pltpu.sync_copy(rcv_hbm.at[wid], idx)

        def zrow(r, carry):
            for j in range(_LAT // 16):
                b0[r, pl.ds(j * 16, 16)] = zero16
            return carry

        lax.fori_loop(0, _C, zrow, 0)
        my_r0 = s * _RPT

        def zchunk(i, carry):
            pltpu.sync_copy(b0, acc.at[pl.ds(my_r0 + i * _C, _C)])
            return carry

        lax.fori_loop(0, _RPT // _C, zchunk, 0)
        plsc.subcore_barrier()

        def issue(j, b, sem):
            row0 = pl.multiple_of(base + j * _C, 8)
            pltpu.async_copy(ne_hbm.at[pl.ds(row0, _C)], b, sem)

        def drain(b, sem):
            pltpu.make_async_copy(ne_hbm.at[pl.ds(base, _C)], b, sem).wait()

        issue(0, b0, r0sem)

        def body(i, carry):
            j1 = 2 * i + 1
            issue(j1, b1, r1sem)
            drain(b0, r0sem)
            pltpu.sync_copy(b0, acc.at[idx.at[2 * i]], add=True)
            issue(j1 + 1, b0, r0sem)
            drain(b1, r1sem)
            pltpu.sync_copy(b1, acc.at[idx.at[j1]], add=True)
            return carry

        lax.fori_loop(0, (_NCH - 1) // 2, body, 0)
        drain(b0, r0sem)
        pltpu.sync_copy(b0, acc.at[idx.at[_NCH - 1]], add=True)
        plsc.subcore_barrier()

        def wchunk(i, carry):
            r0 = my_r0 + i * _ZR
            pltpu.sync_copy(acc.at[pl.ds(r0, _ZR)], out_hbm.at[c, pl.ds(r0, _ZR)])
            return carry

        lax.fori_loop(0, _RPT // _ZR, wchunk, 0)

    return k


def _sc_gather_add(sw, rw, snd3, rcv3):
    return _gather_add_kernel()(sw, rw, snd3, rcv3)


def _sc_scatter_partials(ne, rcv3):
    return _scatter_add_kernel()(ne, rcv3)


def _r(b):
    return b.reshape(1, -1)


def kernel(node_features, edge_features, senders, receivers, params):
    p = params
    (nw1, nb1), (nw2, nb2), (nw3, nb3) = p['node_enc']['mlp']
    ng, nb = p['node_enc']['ln']
    (ew1, eb1), (ew2, eb2), (ew3, eb3) = p['edge_enc']['mlp']
    eg, eb = p['edge_enc']['ln']
    (d1, f1), (d2, f2), (d3, f3) = p['dec']

    blocks = []
    for bp in p['blocks']:
        (w1e, b1e), (w2e, b2e), (w3e, b3e) = bp['edge']['mlp']
        (w1n, b1n), (w2n, b2n), (w3n, b3n) = bp['node']['mlp']
        blocks.append(dict(
            ws=w1e[:_LAT], wr=w1e[_LAT:2 * _LAT], we=w1e[2 * _LAT:],
            b1e=_r(b1e), w2e=w2e, b2e=_r(b2e), w3e=w3e, b3e=_r(b3e),
            ge=_r(bp['edge']['ln'][0]), be=_r(bp['edge']['ln'][1]),
            wn=w1n[:_LAT], wa=w1n[_LAT:], b1n=_r(b1n),
            w2n=w2n, b2n=_r(b2n), w3n=w3n, b3n=_r(b3n),
            gn=_r(bp['node']['ln'][0]), bn=_r(bp['node']['ln'][1]),
        ))

    d3p = jnp.zeros((_LAT, _LAT), jnp.float32).at[:, :_OUT].set(d3)
    f3p = jnp.zeros((1, _LAT), jnp.float32).at[:, :_OUT].set(_r(f3))

    node_lat, sw, rw = _tc_call(
        _node_enc_body, [node_features],
        [nw1, _r(nb1), nw2, _r(nb2), nw3, _r(nb3), _r(ng), _r(nb),
         blocks[0]['ws'], blocks[0]['wr'], blocks[0]['b1e']],
        [(_N, _LAT)] * 3, _RN, _N)

    eft = edge_features.T
    ew_args = [ew1, _r(eb1), ew2, _r(eb2), ew3, _r(eb3), _r(eg), _r(eb)]
    edge_lat = pl.pallas_call(
        _edge_enc_body,
        grid=(_E // _RE,),
        in_specs=([pl.BlockSpec((4, _RE), lambda i: (0, i))]
                  + [_full_spec(a) for a in ew_args]),
        out_specs=_row_spec(_RE, _LAT),
        out_shape=jax.ShapeDtypeStruct((_E, _LAT), jnp.float32),
        compiler_params=pltpu.CompilerParams(
            dimension_semantics=("parallel",)),
    )(eft, *ew_args)

    snd3 = senders.reshape(_NW, _NCH, _C)
    rcv3 = receivers.reshape(_NW, _NCH, _C)
    for s, bl in enumerate(blocks):
        g = _sc_gather_add(sw, rw, snd3, rcv3)
        ew = [bl['we'], bl['w2e'], bl['b2e'], bl['w3e'], bl['b3e'],
              bl['ge'], bl['be']]
        if s < len(blocks) - 1:
            ne, edge_lat = _tc_call(
                _edge_step_body, [g, edge_lat], ew, [(_E, _LAT)] * 2, _RE, _E)
        else:
            ne = _tc_call(
                _edge_last_body, [g, edge_lat], ew, [(_E, _LAT)], _RE, _E)
        parts = _sc_scatter_partials(ne, rcv3)
        if s < len(blocks) - 1:
            nxt = blocks[s + 1]
            node_lat, sw, rw = _tc_call(
                _node_step_body, [node_lat, parts],
                [bl['wn'], bl['wa'], bl['b1n'], bl['w2n'], bl['b2n'],
                 bl['w3n'], bl['b3n'], bl['gn'], bl['bn'],
                 nxt['ws'], nxt['wr'], nxt['b1e']],
                [(_N, _LAT)] * 3, _RN, _N)
        else:
            dec = _tc_call(
                _node_final_body, [node_lat, parts],
                [bl['wn'], bl['wa'], bl['b1n'], bl['w2n'], bl['b2n'],
                 bl['w3n'], bl['b3n'], bl['gn'], bl['bn'],
                 d1, _r(f1), d2, _r(f2), d3p, f3p],
                [(_N, _LAT)], _RN, _N)
    return dec[:, :_OUT]


# speedup vs baseline: 4.2124x; 1.2115x over previous
import functools

import jax
import jax.numpy as jnp
from jax import lax
from jax.experimental import pallas as pl
from jax.experimental.pallas import tpu as pltpu
from jax.experimental.pallas import tpu_sc as plsc

_N = 10000
_E = 320000
_LAT = 128
_OUT = 3
_EPS = 1e-5

_EH = _E // 2
_RE = 1280
_RN = 2000
_NC, _NS = 2, 16
_NW = _NC * _NS
_EPW = _EH // _NW
_C = 40
_NPAD = 10240
_RPT = _NPAD // _NS
_ZR = 128


def _ln(y, g, b):
    mu = jnp.mean(y, axis=-1, keepdims=True)
    yc = y - mu
    var = jnp.mean(yc * yc, axis=-1, keepdims=True)
    return yc / jnp.sqrt(var + _EPS) * g + b


def _dot(x, w):
    return jnp.dot(x, w, preferred_element_type=jnp.float32)


def _node_enc_body(x, w1, b1, w2, b2, w3, b3, g, bb, ws, wr, b1e,
                   o_lat, o_sw, o_rw):
    h = jnp.maximum(_dot(x[...], w1[...]) + b1[...], 0.0)
    h = jnp.maximum(_dot(h, w2[...]) + b2[...], 0.0)
    y = _ln(_dot(h, w3[...]) + b3[...], g[...], bb[...])
    o_lat[...] = y
    o_sw[...] = _dot(y, ws[...]) + b1e[...]
    o_rw[...] = _dot(y, wr[...])


def _edge_enc_body(xt, w1, b1, w2, b2, w3, b3, g, bb, o_lat):
    h = lax.dot_general(xt[...], w1[...], (((0,), (0,)), ((), ())),
                        preferred_element_type=jnp.float32)
    h = jnp.maximum(h + b1[...], 0.0)
    h = jnp.maximum(_dot(h, w2[...]) + b2[...], 0.0)
    o_lat[...] = _ln(_dot(h, w3[...]) + b3[...], g[...], bb[...])


def _edge_step_body(g_in, elat, we, w2, b2, w3, b3, lg, lb, o_ne, o_eout):
    e = elat[...]
    h = jnp.maximum(g_in[...] + _dot(e, we[...]), 0.0)
    h = jnp.maximum(_dot(h, w2[...]) + b2[...], 0.0)
    ne = _ln(_dot(h, w3[...]) + b3[...], lg[...], lb[...])
    o_ne[...] = ne
    o_eout[...] = e + ne


def _edge_last_body(g_in, elat, we, w2, b2, w3, b3, lg, lb, o_ne):
    h = jnp.maximum(g_in[...] + _dot(elat[...], we[...]), 0.0)
    h = jnp.maximum(_dot(h, w2[...]) + b2[...], 0.0)
    o_ne[...] = _ln(_dot(h, w3[...]) + b3[...], lg[...], lb[...])


def _node_step_body(nlat, pa, pb, wn, wa, b1, w2, b2, w3, b3, g, bb,
                    ws, wr, b1e, o_lat, o_sw, o_rw):
    n = nlat[...]
    agg = (pa[0] + pa[1]) + (pb[0] + pb[1])
    h = jnp.maximum(_dot(n, wn[...]) + _dot(agg, wa[...]) + b1[...], 0.0)
    h = jnp.maximum(_dot(h, w2[...]) + b2[...], 0.0)
    y = _ln(_dot(h, w3[...]) + b3[...], g[...], bb[...])
    no = n + y
    o_lat[...] = no
    o_sw[...] = _dot(no, ws[...]) + b1e[...]
    o_rw[...] = _dot(no, wr[...])


def _node_final_body(nlat, pa, pb, wn, wa, b1, w2, b2, w3, b3, g, bb,
                     d1, e1, d2, e2, d3, e3, o_dec):
    n = nlat[...]
    agg = (pa[0] + pa[1]) + (pb[0] + pb[1])
    h = jnp.maximum(_dot(n, wn[...]) + _dot(agg, wa[...]) + b1[...], 0.0)
    h = jnp.maximum(_dot(h, w2[...]) + b2[...], 0.0)
    y = _ln(_dot(h, w3[...]) + b3[...], g[...], bb[...])
    no = n + y
    h = jnp.maximum(_dot(no, d1[...]) + e1[...], 0.0)
    h = jnp.maximum(_dot(h, d2[...]) + e2[...], 0.0)
    o_dec[...] = _dot(h, d3[...]) + e3[...]


def _full_spec(a):
    nd = a.ndim
    return pl.BlockSpec(a.shape, lambda i, _nd=nd: (0,) * _nd)


def _row_spec(rows, cols):
    return pl.BlockSpec((rows, cols), lambda i: (i, 0))


def _tc_call(body, row_args, weight_args, out_shapes, rows, total_rows):
    grid = (total_rows // rows,)
    in_specs = [
        (pl.BlockSpec((_NC, rows, a.shape[-1]), lambda i: (0, i, 0))
         if a.ndim == 3 else _row_spec(rows, a.shape[-1]))
        for a in row_args
    ]
    in_specs += [_full_spec(a) for a in weight_args]
    out_specs = [_row_spec(rows, s[-1]) for s in out_shapes]
    outs = pl.pallas_call(
        body,
        grid=grid,
        in_specs=in_specs,
        out_specs=out_specs if len(out_specs) > 1 else out_specs[0],
        out_shape=([jax.ShapeDtypeStruct(s, jnp.float32) for s in out_shapes]
                   if len(out_shapes) > 1
                   else jax.ShapeDtypeStruct(out_shapes[0], jnp.float32)),
        compiler_params=pltpu.CompilerParams(
            dimension_semantics=("parallel",)),
    )(*row_args, *weight_args)
    return outs


@functools.cache
def _sc_mesh():
    return plsc.VectorSubcoreMesh(core_axis_name="c", subcore_axis_name="s",
                                  num_cores=_NC, num_subcores=_NS)


_NCH = _EPW // _C


@functools.cache
def _gather_add_kernel():
    @functools.partial(
        pl.kernel,
        out_type=jax.ShapeDtypeStruct((_EH, _LAT), jnp.float32),
        mesh=_sc_mesh(),
        scratch_types=[
            pltpu.VMEM((_NCH, _C), jnp.int32),
            pltpu.VMEM((_NCH, _C), jnp.int32),
            pltpu.VMEM((_C, _LAT), jnp.float32),
            pltpu.VMEM((_C, _LAT), jnp.float32),
            pltpu.VMEM((_C, _LAT), jnp.float32),
            pltpu.VMEM((_C, _LAT), jnp.float32),
            pltpu.VMEM((_C, _LAT), jnp.float32),
            pltpu.VMEM((_C, _LAT), jnp.float32),
            pltpu.SemaphoreType.DMA,
            pltpu.SemaphoreType.DMA,
            pltpu.SemaphoreType.DMA,
            pltpu.SemaphoreType.DMA,
        ],
    )
    def k(sw_hbm, rw_hbm, snd_hbm, rcv_hbm, out_hbm,
          idx_s, idx_r, bs0, br0, bs1, br1, ob0, ob1, g0, g1, w0, w1):
        c = lax.axis_index("c")
        s = lax.axis_index("s")
        wid = c * _NS + s
        base = wid * _EPW

        pltpu.sync_copy(snd_hbm.at[wid], idx_s)
        pltpu.sync_copy(rcv_hbm.at[wid], idx_r)

        def issue(j, bs, br, sem):
            pltpu.async_copy(sw_hbm.at[idx_s.at[j]], bs, sem)
            pltpu.async_copy(rw_hbm.at[idx_r.at[j]], br, sem)

        def drain_g(bs, br, sem):
            pltpu.make_async_copy(sw_hbm.at[idx_s.at[0]], bs, sem).wait()
            pltpu.make_async_copy(rw_hbm.at[idx_r.at[0]], br, sem).wait()

        def add(bs, br, ob):
            def addrow(r, carry2):
                for q in range(_LAT // 16):
                    sl = pl.ds(q * 16, 16)
                    ob[r, sl] = bs[r, sl] + br[r, sl]
                return carry2

            lax.fori_loop(0, _C, addrow, 0, unroll=2)

        def issue_w(j, ob, wsem):
            row0 = pl.multiple_of(base + j * _C, 8)
            pltpu.async_copy(ob, out_hbm.at[pl.ds(row0, _C)], wsem)

        def drain_w(ob, wsem):
            pltpu.make_async_copy(ob, out_hbm.at[pl.ds(base, _C)], wsem).wait()

        issue(0, bs0, br0, g0)
        issue(1, bs1, br1, g1)
        nb = (_NCH - 1) // 2

        def body(i, carry):
            j0 = 2 * i
            drain_g(bs0, br0, g0)

            @pl.when(i > 0)
            def _():
                drain_w(ob0, w0)

            add(bs0, br0, ob0)

            @pl.when(j0 + 2 < _NCH)
            def _():
                issue(j0 + 2, bs0, br0, g0)

            issue_w(j0, ob0, w0)

            drain_g(bs1, br1, g1)

            @pl.when(i > 0)
            def _():
                drain_w(ob1, w1)

            add(bs1, br1, ob1)

            @pl.when(j0 + 3 < _NCH)
            def _():
                issue(j0 + 3, bs1, br1, g1)

            issue_w(j0 + 1, ob1, w1)
            return carry

        lax.fori_loop(0, nb, body, 0)
        drain_g(bs0, br0, g0)
        drain_w(ob0, w0)
        add(bs0, br0, ob0)
        issue_w(_NCH - 1, ob0, w0)
        drain_w(ob0, w0)
        drain_w(ob1, w1)

    return k


@functools.cache
def _scatter_add_kernel():
    @functools.partial(
        pl.kernel,
        out_type=jax.ShapeDtypeStruct((_NC, _NPAD, _LAT), jnp.float32),
        mesh=_sc_mesh(),
        scratch_types=[
            pltpu.VMEM((_NCH, _C), jnp.int32),
            pltpu.VMEM((_C, _LAT), jnp.float32),
            pltpu.VMEM((_C, _LAT), jnp.float32),
            pltpu.VMEM_SHARED((_NPAD, _LAT), jnp.float32),
            pltpu.SemaphoreType.DMA,
            pltpu.SemaphoreType.DMA,
        ],
    )
    def k(ne_hbm, rcv_hbm, out_hbm, idx, b0, b1, acc, r0sem, r1sem):
        c = lax.axis_index("c")
        s = lax.axis_index("s")
        wid = c * _NS + s
        base = wid * _EPW
        zero16 = jnp.zeros((16,), jnp.float32)

        pltpu.sync_copy(rcv_hbm.at[wid], idx)

        def zrow(r, carry):
            for j in range(_LAT // 16):
                b0[r, pl.ds(j * 16, 16)] = zero16
            return carry

        lax.fori_loop(0, _C, zrow, 0)
        my_r0 = s * _RPT

        def zchunk(i, carry):
            pltpu.sync_copy(b0, acc.at[pl.ds(my_r0 + i * _C, _C)])
            return carry

        lax.fori_loop(0, _RPT // _C, zchunk, 0)
        plsc.subcore_barrier()

        def issue(j, b, sem):
            row0 = pl.multiple_of(base + j * _C, 8)
            pltpu.async_copy(ne_hbm.at[pl.ds(row0, _C)], b, sem)

        def drain(b, sem):
            pltpu.make_async_copy(ne_hbm.at[pl.ds(base, _C)], b, sem).wait()

        issue(0, b0, r0sem)

        def body(i, carry):
            j1 = 2 * i + 1
            issue(j1, b1, r1sem)
            drain(b0, r0sem)
            pltpu.sync_copy(b0, acc.at[idx.at[2 * i]], add=True)
            issue(j1 + 1, b0, r0sem)
            drain(b1, r1sem)
            pltpu.sync_copy(b1, acc.at[idx.at[j1]], add=True)
            return carry

        lax.fori_loop(0, (_NCH - 1) // 2, body, 0)
        drain(b0, r0sem)
        pltpu.sync_copy(b0, acc.at[idx.at[_NCH - 1]], add=True)
        plsc.subcore_barrier()

        def wchunk(i, carry):
            r0 = my_r0 + i * _ZR
            pltpu.sync_copy(acc.at[pl.ds(r0, _ZR)], out_hbm.at[c, pl.ds(r0, _ZR)])
            return carry

        lax.fori_loop(0, _RPT // _ZR, wchunk, 0)

    return k


def _sc_gather_add(sw, rw, snd3, rcv3):
    return _gather_add_kernel()(sw, rw, snd3, rcv3)


def _sc_scatter_partials(ne, rcv3):
    return _scatter_add_kernel()(ne, rcv3)


def _r(b):
    return b.reshape(1, -1)


def kernel(node_features, edge_features, senders, receivers, params):
    p = params
    (nw1, nb1), (nw2, nb2), (nw3, nb3) = p['node_enc']['mlp']
    ng, nb = p['node_enc']['ln']
    (ew1, eb1), (ew2, eb2), (ew3, eb3) = p['edge_enc']['mlp']
    eg, eb = p['edge_enc']['ln']
    (d1, f1), (d2, f2), (d3, f3) = p['dec']

    blocks = []
    for bp in p['blocks']:
        (w1e, b1e), (w2e, b2e), (w3e, b3e) = bp['edge']['mlp']
        (w1n, b1n), (w2n, b2n), (w3n, b3n) = bp['node']['mlp']
        blocks.append(dict(
            ws=w1e[:_LAT], wr=w1e[_LAT:2 * _LAT], we=w1e[2 * _LAT:],
            b1e=_r(b1e), w2e=w2e, b2e=_r(b2e), w3e=w3e, b3e=_r(b3e),
            ge=_r(bp['edge']['ln'][0]), be=_r(bp['edge']['ln'][1]),
            wn=w1n[:_LAT], wa=w1n[_LAT:], b1n=_r(b1n),
            w2n=w2n, b2n=_r(b2n), w3n=w3n, b3n=_r(b3n),
            gn=_r(bp['node']['ln'][0]), bn=_r(bp['node']['ln'][1]),
        ))

    d3p = jnp.zeros((_LAT, _LAT), jnp.float32).at[:, :_OUT].set(d3)
    f3p = jnp.zeros((1, _LAT), jnp.float32).at[:, :_OUT].set(_r(f3))

    node_lat, sw, rw = _tc_call(
        _node_enc_body, [node_features],
        [nw1, _r(nb1), nw2, _r(nb2), nw3, _r(nb3), _r(ng), _r(nb),
         blocks[0]['ws'], blocks[0]['wr'], blocks[0]['b1e']],
        [(_N, _LAT)] * 3, _RN, _N)

    ew_args = [ew1, _r(eb1), ew2, _r(eb2), ew3, _r(eb3), _r(eg), _r(eb)]

    def edge_enc(eft_half):
        return pl.pallas_call(
            _edge_enc_body,
            grid=(_EH // _RE,),
            in_specs=([pl.BlockSpec((4, _RE), lambda i: (0, i))]
                      + [_full_spec(a) for a in ew_args]),
            out_specs=_row_spec(_RE, _LAT),
            out_shape=jax.ShapeDtypeStruct((_EH, _LAT), jnp.float32),
            compiler_params=pltpu.CompilerParams(
                dimension_semantics=("parallel",)),
        )(eft_half, *ew_args)

    elat = [edge_enc(edge_features[:_EH].T), edge_enc(edge_features[_EH:].T)]
    idx_s = [senders[:_EH].reshape(_NW, -1, _C),
             senders[_EH:].reshape(_NW, -1, _C)]
    idx_r = [receivers[:_EH].reshape(_NW, -1, _C),
             receivers[_EH:].reshape(_NW, -1, _C)]

    for s, bl in enumerate(blocks):
        ew = [bl['we'], bl['w2e'], bl['b2e'], bl['w3e'], bl['b3e'],
              bl['ge'], bl['be']]
        g = [_sc_gather_add(sw, rw, idx_s[h], idx_r[h]) for h in (0, 1)]
        ne = [None, None]
        parts = [None, None]
        for h in (0, 1):
            if s < len(blocks) - 1:
                ne[h], elat[h] = _tc_call(
                    _edge_step_body, [g[h], elat[h]], ew,
                    [(_EH, _LAT)] * 2, _RE, _EH)
            else:
                ne[h] = _tc_call(
                    _edge_last_body, [g[h], elat[h]], ew,
                    [(_EH, _LAT)], _RE, _EH)
            parts[h] = _sc_scatter_partials(ne[h], idx_r[h])
        if s < len(blocks) - 1:
            nxt = blocks[s + 1]
            node_lat, sw, rw = _tc_call(
                _node_step_body, [node_lat, parts[0], parts[1]],
                [bl['wn'], bl['wa'], bl['b1n'], bl['w2n'], bl['b2n'],
                 bl['w3n'], bl['b3n'], bl['gn'], bl['bn'],
                 nxt['ws'], nxt['wr'], nxt['b1e']],
                [(_N, _LAT)] * 3, _RN, _N)
        else:
            dec = _tc_call(
                _node_final_body, [node_lat, parts[0], parts[1]],
                [bl['wn'], bl['wa'], bl['b1n'], bl['w2n'], bl['b2n'],
                 bl['w3n'], bl['b3n'], bl['gn'], bl['bn'],
                 d1, _r(f1), d2, _r(f2), d3p, f3p],
                [(_N, _LAT)], _RN, _N)
    return dec[:, :_OUT]


# speedup vs baseline: 4.2408x; 1.0068x over previous
import functools

import jax
import jax.numpy as jnp
from jax import lax
from jax.experimental import pallas as pl
from jax.experimental.pallas import tpu as pltpu
from jax.experimental.pallas import tpu_sc as plsc

_N = 10000
_E = 320000
_LAT = 128
_OUT = 3
_EPS = 1e-5

_EH = _E // 2
_RE = 1280
_RN = 2000
_NC, _NS = 2, 16
_NW = _NC * _NS
_EPW = _EH // _NW
_C = 40
_NPAD = 10240
_RPT = _NPAD // _NS
_ZR = 128


def _ln(y, g, b):
    mu = jnp.mean(y, axis=-1, keepdims=True)
    yc = y - mu
    var = jnp.mean(yc * yc, axis=-1, keepdims=True)
    return yc / jnp.sqrt(var + _EPS) * g + b


def _dot(x, w):
    return jnp.dot(x, w, preferred_element_type=jnp.float32)


def _node_enc_body(x, w1, b1, w2, b2, w3, b3, g, bb, ws, wr, b1e,
                   o_lat, o_sw, o_rw):
    h = jnp.maximum(_dot(x[...], w1[...]) + b1[...], 0.0)
    h = jnp.maximum(_dot(h, w2[...]) + b2[...], 0.0)
    y = _ln(_dot(h, w3[...]) + b3[...], g[...], bb[...])
    o_lat[...] = y
    o_sw[...] = _dot(y, ws[...]) + b1e[...]
    o_rw[...] = _dot(y, wr[...])


def _edge_enc_body(xt, w1, b1, w2, b2, w3, b3, g, bb, o_lat):
    h = lax.dot_general(xt[...], w1[...], (((0,), (0,)), ((), ())),
                        preferred_element_type=jnp.float32)
    h = jnp.maximum(h + b1[...], 0.0)
    h = jnp.maximum(_dot(h, w2[...]) + b2[...], 0.0)
    o_lat[...] = _ln(_dot(h, w3[...]) + b3[...], g[...], bb[...])


def _edge_step_body(g_in, elat, we, w2, b2, w3, b3, lg, lb, o_ne, o_eout):
    e = elat[...]
    h = jnp.maximum(g_in[...] + _dot(e, we[...]), 0.0)
    h = jnp.maximum(_dot(h, w2[...]) + b2[...], 0.0)
    ne = _ln(_dot(h, w3[...]) + b3[...], lg[...], lb[...])
    o_ne[...] = ne
    o_eout[...] = e + ne


def _edge_last_body(g_in, elat, we, w2, b2, w3, b3, lg, lb, o_ne):
    h = jnp.maximum(g_in[...] + _dot(elat[...], we[...]), 0.0)
    h = jnp.maximum(_dot(h, w2[...]) + b2[...], 0.0)
    o_ne[...] = _ln(_dot(h, w3[...]) + b3[...], lg[...], lb[...])


def _node_step_body(nlat, pa, pb, wn, wa, b1, w2, b2, w3, b3, g, bb,
                    ws, wr, b1e, o_lat, o_sw, o_rw):
    n = nlat[...]
    agg = (pa[0] + pa[1]) + (pb[0] + pb[1])
    h = jnp.maximum(_dot(n, wn[...]) + _dot(agg, wa[...]) + b1[...], 0.0)
    h = jnp.maximum(_dot(h, w2[...]) + b2[...], 0.0)
    y = _ln(_dot(h, w3[...]) + b3[...], g[...], bb[...])
    no = n + y
    o_lat[...] = no
    o_sw[...] = _dot(no, ws[...]) + b1e[...]
    o_rw[...] = _dot(no, wr[...])


def _node_final_body(nlat, pa, pb, wn, wa, b1, w2, b2, w3, b3, g, bb,
                     d1, e1, d2, e2, d3, e3, o_dec):
    n = nlat[...]
    agg = (pa[0] + pa[1]) + (pb[0] + pb[1])
    h = jnp.maximum(_dot(n, wn[...]) + _dot(agg, wa[...]) + b1[...], 0.0)
    h = jnp.maximum(_dot(h, w2[...]) + b2[...], 0.0)
    y = _ln(_dot(h, w3[...]) + b3[...], g[...], bb[...])
    no = n + y
    h = jnp.maximum(_dot(no, d1[...]) + e1[...], 0.0)
    h = jnp.maximum(_dot(h, d2[...]) + e2[...], 0.0)
    o_dec[...] = _dot(h, d3[...]) + e3[...]


def _full_spec(a):
    nd = a.ndim
    return pl.BlockSpec(a.shape, lambda i, _nd=nd: (0,) * _nd)


def _row_spec(rows, cols):
    return pl.BlockSpec((rows, cols), lambda i: (i, 0))


def _tc_call(body, row_args, weight_args, out_shapes, rows, total_rows):
    grid = (total_rows // rows,)
    in_specs = [
        (pl.BlockSpec((_NC, rows, a.shape[-1]), lambda i: (0, i, 0))
         if a.ndim == 3 else _row_spec(rows, a.shape[-1]))
        for a in row_args
    ]
    in_specs += [_full_spec(a) for a in weight_args]
    out_specs = [_row_spec(rows, s[-1]) for s in out_shapes]
    outs = pl.pallas_call(
        body,
        grid=grid,
        in_specs=in_specs,
        out_specs=out_specs if len(out_specs) > 1 else out_specs[0],
        out_shape=([jax.ShapeDtypeStruct(s, jnp.float32) for s in out_shapes]
                   if len(out_shapes) > 1
                   else jax.ShapeDtypeStruct(out_shapes[0], jnp.float32)),
        compiler_params=pltpu.CompilerParams(
            dimension_semantics=("parallel",)),
    )(*row_args, *weight_args)
    return outs


@functools.cache
def _sc_mesh():
    return plsc.VectorSubcoreMesh(core_axis_name="c", subcore_axis_name="s",
                                  num_cores=_NC, num_subcores=_NS)


_NCH = _EPW // _C


_GC = 80
_NGF = _EPW // _GC
_TAIL = _EPW - _NGF * _GC


@functools.cache
def _gather_add_kernel():
    @functools.partial(
        pl.kernel,
        out_type=jax.ShapeDtypeStruct((_EH, _LAT), jnp.float32),
        mesh=_sc_mesh(),
        scratch_types=[
            pltpu.VMEM((_EPW,), jnp.int32),
            pltpu.VMEM((_EPW,), jnp.int32),
            pltpu.VMEM((_GC, _LAT), jnp.float32),
            pltpu.VMEM((_GC, _LAT), jnp.float32),
            pltpu.VMEM((_GC, _LAT), jnp.float32),
            pltpu.VMEM((_GC, _LAT), jnp.float32),
            pltpu.VMEM((_GC, _LAT), jnp.float32),
            pltpu.VMEM((_GC, _LAT), jnp.float32),
            pltpu.SemaphoreType.DMA,
            pltpu.SemaphoreType.DMA,
            pltpu.SemaphoreType.DMA,
            pltpu.SemaphoreType.DMA,
        ],
    )
    def k(sw_hbm, rw_hbm, snd_hbm, rcv_hbm, out_hbm,
          idx_s, idx_r, bs0, br0, bs1, br1, ob0, ob1, g0, g1, w0, w1):
        c = lax.axis_index("c")
        s = lax.axis_index("s")
        wid = c * _NS + s
        base = wid * _EPW

        pltpu.sync_copy(snd_hbm.at[wid], idx_s)
        pltpu.sync_copy(rcv_hbm.at[wid], idx_r)

        def issue(j, n, bs, br, sem):
            off = pl.multiple_of(j * _GC, 8)
            pltpu.async_copy(sw_hbm.at[idx_s.at[pl.ds(off, n)]],
                             bs.at[pl.ds(0, n)], sem)
            pltpu.async_copy(rw_hbm.at[idx_r.at[pl.ds(off, n)]],
                             br.at[pl.ds(0, n)], sem)

        def drain_g(n, bs, br, sem):
            pltpu.make_async_copy(sw_hbm.at[idx_s.at[pl.ds(0, n)]],
                                  bs.at[pl.ds(0, n)], sem).wait()
            pltpu.make_async_copy(rw_hbm.at[idx_r.at[pl.ds(0, n)]],
                                  br.at[pl.ds(0, n)], sem).wait()

        def add(n, bs, br, ob):
            def addrow(r, carry2):
                for q in range(_LAT // 16):
                    sl = pl.ds(q * 16, 16)
                    ob[r, sl] = bs[r, sl] + br[r, sl]
                return carry2

            lax.fori_loop(0, n, addrow, 0, unroll=2)

        def issue_w(j, n, ob, wsem):
            row0 = pl.multiple_of(base + j * _GC, 8)
            pltpu.async_copy(ob.at[pl.ds(0, n)],
                             out_hbm.at[pl.ds(row0, n)], wsem)

        def drain_w(n, ob, wsem):
            pltpu.make_async_copy(ob.at[pl.ds(0, n)],
                                  out_hbm.at[pl.ds(base, n)], wsem).wait()

        issue(0, _GC, bs0, br0, g0)
        issue(1, _GC, bs1, br1, g1)

        def body(i, carry):
            j0 = 2 * i
            drain_g(_GC, bs0, br0, g0)

            @pl.when(i > 0)
            def _():
                drain_w(_GC, ob0, w0)

            add(_GC, bs0, br0, ob0)

            @pl.when(j0 + 2 < _NGF)
            def _():
                issue(j0 + 2, _GC, bs0, br0, g0)

            @pl.when(j0 + 2 == _NGF)
            def _():
                issue(j0 + 2, _TAIL, bs0, br0, g0)

            issue_w(j0, _GC, ob0, w0)

            drain_g(_GC, bs1, br1, g1)

            @pl.when(i > 0)
            def _():
                drain_w(_GC, ob1, w1)

            add(_GC, bs1, br1, ob1)

            @pl.when(j0 + 3 < _NGF)
            def _():
                issue(j0 + 3, _GC, bs1, br1, g1)

            @pl.when(j0 + 3 == _NGF)
            def _():
                issue(j0 + 3, _TAIL, bs1, br1, g1)

            issue_w(j0 + 1, _GC, ob1, w1)
            return carry

        lax.fori_loop(0, _NGF // 2, body, 0)
        drain_g(_TAIL, bs0, br0, g0)
        drain_w(_GC, ob0, w0)
        add(_TAIL, bs0, br0, ob0)
        issue_w(_NGF, _TAIL, ob0, w0)
        drain_w(_TAIL, ob0, w0)
        drain_w(_GC, ob1, w1)

    return k


@functools.cache
def _scatter_add_kernel():
    @functools.partial(
        pl.kernel,
        out_type=jax.ShapeDtypeStruct((_NC, _NPAD, _LAT), jnp.float32),
        mesh=_sc_mesh(),
        scratch_types=[
            pltpu.VMEM((_NCH, _C), jnp.int32),
            pltpu.VMEM((_C, _LAT), jnp.float32),
            pltpu.VMEM((_C, _LAT), jnp.float32),
            pltpu.VMEM_SHARED((_NPAD, _LAT), jnp.float32),
            pltpu.SemaphoreType.DMA,
            pltpu.SemaphoreType.DMA,
        ],
    )
    def k(ne_hbm, rcv_hbm, out_hbm, idx, b0, b1, acc, r0sem, r1sem):
        c = lax.axis_index("c")
        s = lax.axis_index("s")
        wid = c * _NS + s
        base = wid * _EPW
        zero16 = jnp.zeros((16,), jnp.float32)

        pltpu.sync_copy(rcv_hbm.at[wid], idx)

        def zrow(r, carry):
            for j in range(_LAT // 16):
                b0[r, pl.ds(j * 16, 16)] = zero16
            return carry

        lax.fori_loop(0, _C, zrow, 0)
        my_r0 = s * _RPT

        def zchunk(i, carry):
            pltpu.sync_copy(b0, acc.at[pl.ds(my_r0 + i * _C, _C)])
            return carry

        lax.fori_loop(0, _RPT // _C, zchunk, 0)
        plsc.subcore_barrier()

        def issue(j, b, sem):
            row0 = pl.multiple_of(base + j * _C, 8)
            pltpu.async_copy(ne_hbm.at[pl.ds(row0, _C)], b, sem)

        def drain(b, sem):
            pltpu.make_async_copy(ne_hbm.at[pl.ds(base, _C)], b, sem).wait()

        issue(0, b0, r0sem)

        def body(i, carry):
            j1 = 2 * i + 1
            issue(j1, b1, r1sem)
            drain(b0, r0sem)
            pltpu.sync_copy(b0, acc.at[idx.at[2 * i]], add=True)
            issue(j1 + 1, b0, r0sem)
            drain(b1, r1sem)
            pltpu.sync_copy(b1, acc.at[idx.at[j1]], add=True)
            return carry

        lax.fori_loop(0, (_NCH - 1) // 2, body, 0)
        drain(b0, r0sem)
        pltpu.sync_copy(b0, acc.at[idx.at[_NCH - 1]], add=True)
        plsc.subcore_barrier()

        def wchunk(i, carry):
            r0 = my_r0 + i * _ZR
            pltpu.sync_copy(acc.at[pl.ds(r0, _ZR)], out_hbm.at[c, pl.ds(r0, _ZR)])
            return carry

        lax.fori_loop(0, _RPT // _ZR, wchunk, 0)

    return k


def _sc_gather_add(sw, rw, snd3, rcv3):
    return _gather_add_kernel()(sw, rw, snd3, rcv3)


def _sc_scatter_partials(ne, rcv3):
    return _scatter_add_kernel()(ne, rcv3)


def _r(b):
    return b.reshape(1, -1)


def kernel(node_features, edge_features, senders, receivers, params):
    p = params
    (nw1, nb1), (nw2, nb2), (nw3, nb3) = p['node_enc']['mlp']
    ng, nb = p['node_enc']['ln']
    (ew1, eb1), (ew2, eb2), (ew3, eb3) = p['edge_enc']['mlp']
    eg, eb = p['edge_enc']['ln']
    (d1, f1), (d2, f2), (d3, f3) = p['dec']

    blocks = []
    for bp in p['blocks']:
        (w1e, b1e), (w2e, b2e), (w3e, b3e) = bp['edge']['mlp']
        (w1n, b1n), (w2n, b2n), (w3n, b3n) = bp['node']['mlp']
        blocks.append(dict(
            ws=w1e[:_LAT], wr=w1e[_LAT:2 * _LAT], we=w1e[2 * _LAT:],
            b1e=_r(b1e), w2e=w2e, b2e=_r(b2e), w3e=w3e, b3e=_r(b3e),
            ge=_r(bp['edge']['ln'][0]), be=_r(bp['edge']['ln'][1]),
            wn=w1n[:_LAT], wa=w1n[_LAT:], b1n=_r(b1n),
            w2n=w2n, b2n=_r(b2n), w3n=w3n, b3n=_r(b3n),
            gn=_r(bp['node']['ln'][0]), bn=_r(bp['node']['ln'][1]),
        ))

    d3p = jnp.zeros((_LAT, _LAT), jnp.float32).at[:, :_OUT].set(d3)
    f3p = jnp.zeros((1, _LAT), jnp.float32).at[:, :_OUT].set(_r(f3))

    node_lat, sw, rw = _tc_call(
        _node_enc_body, [node_features],
        [nw1, _r(nb1), nw2, _r(nb2), nw3, _r(nb3), _r(ng), _r(nb),
         blocks[0]['ws'], blocks[0]['wr'], blocks[0]['b1e']],
        [(_N, _LAT)] * 3, _RN, _N)

    ew_args = [ew1, _r(eb1), ew2, _r(eb2), ew3, _r(eb3), _r(eg), _r(eb)]

    def edge_enc(eft_half):
        return pl.pallas_call(
            _edge_enc_body,
            grid=(_EH // _RE,),
            in_specs=([pl.BlockSpec((4, _RE), lambda i: (0, i))]
                      + [_full_spec(a) for a in ew_args]),
            out_specs=_row_spec(_RE, _LAT),
            out_shape=jax.ShapeDtypeStruct((_EH, _LAT), jnp.float32),
            compiler_params=pltpu.CompilerParams(
                dimension_semantics=("parallel",)),
        )(eft_half, *ew_args)

    elat = [edge_enc(edge_features[:_EH].T), edge_enc(edge_features[_EH:].T)]
    idx_s = [senders[:_EH].reshape(_NW, -1),
             senders[_EH:].reshape(_NW, -1)]
    idx_r = [receivers[:_EH].reshape(_NW, -1),
             receivers[_EH:].reshape(_NW, -1)]
    idx_r3 = [receivers[:_EH].reshape(_NW, -1, _C),
              receivers[_EH:].reshape(_NW, -1, _C)]

    for s, bl in enumerate(blocks):
        ew = [bl['we'], bl['w2e'], bl['b2e'], bl['w3e'], bl['b3e'],
              bl['ge'], bl['be']]
        g = [_sc_gather_add(sw, rw, idx_s[h], idx_r[h]) for h in (0, 1)]
        ne = [None, None]
        parts = [None, None]
        for h in (0, 1):
            if s < len(blocks) - 1:
                ne[h], elat[h] = _tc_call(
                    _edge_step_body, [g[h], elat[h]], ew,
                    [(_EH, _LAT)] * 2, _RE, _EH)
            else:
                ne[h] = _tc_call(
                    _edge_last_body, [g[h], elat[h]], ew,
                    [(_EH, _LAT)], _RE, _EH)
            parts[h] = _sc_scatter_partials(ne[h], idx_r3[h])
        if s < len(blocks) - 1:
            nxt = blocks[s + 1]
            node_lat, sw, rw = _tc_call(
                _node_step_body, [node_lat, parts[0], parts[1]],
                [bl['wn'], bl['wa'], bl['b1n'], bl['w2n'], bl['b2n'],
                 bl['w3n'], bl['b3n'], bl['gn'], bl['bn'],
                 nxt['ws'], nxt['wr'], nxt['b1e']],
                [(_N, _LAT)] * 3, _RN, _N)
        else:
            dec = _tc_call(
                _node_final_body, [node_lat, parts[0], parts[1]],
                [bl['wn'], bl['wa'], bl['b1n'], bl['w2n'], bl['b2n'],
                 bl['w3n'], bl['b3n'], bl['gn'], bl['bn'],
                 d1, _r(f1), d2, _r(f2), d3p, f3p],
                [(_N, _LAT)], _RN, _N)
    return dec[:, :_OUT]


# speedup vs baseline: 4.2430x; 1.0005x over previous
import functools

import jax
import jax.numpy as jnp
from jax import lax
from jax.experimental import pallas as pl
from jax.experimental.pallas import tpu as pltpu
from jax.experimental.pallas import tpu_sc as plsc

_N = 10000
_E = 320000
_LAT = 128
_OUT = 3
_EPS = 1e-5

_EH = _E // 2
_RE = 1280
_RN = 2000
_NC, _NS = 2, 16
_NW = _NC * _NS
_EPW = _EH // _NW
_C = 40
_NPAD = 10240
_RPT = _NPAD // _NS
_ZR = 128


def _ln(y, g, b):
    mu = jnp.mean(y, axis=-1, keepdims=True)
    yc = y - mu
    var = jnp.mean(yc * yc, axis=-1, keepdims=True)
    return yc / jnp.sqrt(var + _EPS) * g + b


def _dot(x, w):
    return jnp.dot(x, w, preferred_element_type=jnp.float32)


def _node_enc_body(x, w1, b1, w2, b2, w3, b3, g, bb, ws, wr, b1e,
                   o_lat, o_sw, o_rw):
    h = jnp.maximum(_dot(x[...], w1[...]) + b1[...], 0.0)
    h = jnp.maximum(_dot(h, w2[...]) + b2[...], 0.0)
    y = _ln(_dot(h, w3[...]) + b3[...], g[...], bb[...])
    o_lat[...] = y
    o_sw[...] = _dot(y, ws[...]) + b1e[...]
    o_rw[...] = _dot(y, wr[...])


def _edge_enc_body(xt, w1, b1, w2, b2, w3, b3, g, bb, o_lat):
    h = lax.dot_general(xt[...], w1[...], (((0,), (0,)), ((), ())),
                        preferred_element_type=jnp.float32)
    h = jnp.maximum(h + b1[...], 0.0)
    h = jnp.maximum(_dot(h, w2[...]) + b2[...], 0.0)
    o_lat[...] = _ln(_dot(h, w3[...]) + b3[...], g[...], bb[...])


def _edge_step_body(g_in, elat, we, w2, b2, w3, b3, lg, lb, o_ne, o_eout):
    e = elat[...]
    h = jnp.maximum(g_in[...] + _dot(e, we[...]), 0.0)
    h = jnp.maximum(_dot(h, w2[...]) + b2[...], 0.0)
    ne = _ln(_dot(h, w3[...]) + b3[...], lg[...], lb[...])
    o_ne[...] = ne
    o_eout[...] = e + ne


def _edge_last_body(g_in, elat, we, w2, b2, w3, b3, lg, lb, o_ne):
    h = jnp.maximum(g_in[...] + _dot(elat[...], we[...]), 0.0)
    h = jnp.maximum(_dot(h, w2[...]) + b2[...], 0.0)
    o_ne[...] = _ln(_dot(h, w3[...]) + b3[...], lg[...], lb[...])


def _node_step_body(nlat, pa, pb, wn, wa, b1, w2, b2, w3, b3, g, bb,
                    ws, wr, b1e, o_lat, o_sw, o_rw):
    n = nlat[...]
    agg = (pa[0] + pa[1]) + (pb[0] + pb[1])
    h = jnp.maximum(_dot(n, wn[...]) + _dot(agg, wa[...]) + b1[...], 0.0)
    h = jnp.maximum(_dot(h, w2[...]) + b2[...], 0.0)
    y = _ln(_dot(h, w3[...]) + b3[...], g[...], bb[...])
    no = n + y
    o_lat[...] = no
    o_sw[...] = _dot(no, ws[...]) + b1e[...]
    o_rw[...] = _dot(no, wr[...])


def _node_final_body(nlat, pa, pb, wn, wa, b1, w2, b2, w3, b3, g, bb,
                     d1, e1, d2, e2, d3, e3, o_dec):
    n = nlat[...]
    agg = (pa[0] + pa[1]) + (pb[0] + pb[1])
    h = jnp.maximum(_dot(n, wn[...]) + _dot(agg, wa[...]) + b1[...], 0.0)
    h = jnp.maximum(_dot(h, w2[...]) + b2[...], 0.0)
    y = _ln(_dot(h, w3[...]) + b3[...], g[...], bb[...])
    no = n + y
    h = jnp.maximum(_dot(no, d1[...]) + e1[...], 0.0)
    h = jnp.maximum(_dot(h, d2[...]) + e2[...], 0.0)
    o_dec[...] = _dot(h, d3[...]) + e3[...]


def _full_spec(a):
    nd = a.ndim
    return pl.BlockSpec(a.shape, lambda i, _nd=nd: (0,) * _nd)


def _row_spec(rows, cols):
    return pl.BlockSpec((rows, cols), lambda i: (i, 0))


def _tc_call(body, row_args, weight_args, out_shapes, rows, total_rows):
    grid = (total_rows // rows,)
    in_specs = [
        (pl.BlockSpec((_NC, rows, a.shape[-1]), lambda i: (0, i, 0))
         if a.ndim == 3 else _row_spec(rows, a.shape[-1]))
        for a in row_args
    ]
    in_specs += [_full_spec(a) for a in weight_args]
    out_specs = [_row_spec(rows, s[-1]) for s in out_shapes]
    outs = pl.pallas_call(
        body,
        grid=grid,
        in_specs=in_specs,
        out_specs=out_specs if len(out_specs) > 1 else out_specs[0],
        out_shape=([jax.ShapeDtypeStruct(s, jnp.float32) for s in out_shapes]
                   if len(out_shapes) > 1
                   else jax.ShapeDtypeStruct(out_shapes[0], jnp.float32)),
        compiler_params=pltpu.CompilerParams(
            dimension_semantics=("parallel",)),
    )(*row_args, *weight_args)
    return outs


@functools.cache
def _sc_mesh():
    return plsc.VectorSubcoreMesh(core_axis_name="c", subcore_axis_name="s",
                                  num_cores=_NC, num_subcores=_NS)


_NCH = _EPW // _C


_GC = 80
_NGF = _EPW // _GC
_TAIL = _EPW - _NGF * _GC


@functools.cache
def _gather_add_kernel():
    @functools.partial(
        pl.kernel,
        out_type=jax.ShapeDtypeStruct((_EH, _LAT), jnp.float32),
        mesh=_sc_mesh(),
        scratch_types=[
            pltpu.VMEM((_EPW,), jnp.int32),
            pltpu.VMEM((_EPW,), jnp.int32),
            pltpu.VMEM((_GC, _LAT), jnp.float32),
            pltpu.VMEM((_GC, _LAT), jnp.float32),
            pltpu.VMEM((_GC, _LAT), jnp.float32),
            pltpu.VMEM((_GC, _LAT), jnp.float32),
            pltpu.VMEM((_GC, _LAT), jnp.float32),
            pltpu.VMEM((_GC, _LAT), jnp.float32),
            pltpu.SemaphoreType.DMA,
            pltpu.SemaphoreType.DMA,
            pltpu.SemaphoreType.DMA,
            pltpu.SemaphoreType.DMA,
        ],
    )
    def k(sw_hbm, rw_hbm, snd_hbm, rcv_hbm, out_hbm,
          idx_s, idx_r, bs0, br0, bs1, br1, ob0, ob1, g0, g1, w0, w1):
        c = lax.axis_index("c")
        s = lax.axis_index("s")
        wid = c * _NS + s
        base = wid * _EPW

        pltpu.sync_copy(snd_hbm.at[wid], idx_s)
        pltpu.sync_copy(rcv_hbm.at[wid], idx_r)

        def issue(j, n, bs, br, sem):
            off = pl.multiple_of(j * _GC, 8)
            pltpu.async_copy(sw_hbm.at[idx_s.at[pl.ds(off, n)]],
                             bs.at[pl.ds(0, n)], sem)
            pltpu.async_copy(rw_hbm.at[idx_r.at[pl.ds(off, n)]],
                             br.at[pl.ds(0, n)], sem)

        def drain_g(n, bs, br, sem):
            pltpu.make_async_copy(sw_hbm.at[idx_s.at[pl.ds(0, n)]],
                                  bs.at[pl.ds(0, n)], sem).wait()
            pltpu.make_async_copy(rw_hbm.at[idx_r.at[pl.ds(0, n)]],
                                  br.at[pl.ds(0, n)], sem).wait()

        def add(n, bs, br, ob):
            def addrow(r, carry2):
                for q in range(_LAT // 16):
                    sl = pl.ds(q * 16, 16)
                    ob[r, sl] = bs[r, sl] + br[r, sl]
                return carry2

            lax.fori_loop(0, n, addrow, 0, unroll=2)

        def issue_w(j, n, ob, wsem):
            row0 = pl.multiple_of(base + j * _GC, 8)
            pltpu.async_copy(ob.at[pl.ds(0, n)],
                             out_hbm.at[pl.ds(row0, n)], wsem)

        def drain_w(n, ob, wsem):
            pltpu.make_async_copy(ob.at[pl.ds(0, n)],
                                  out_hbm.at[pl.ds(base, n)], wsem).wait()

        issue(0, _GC, bs0, br0, g0)
        issue(1, _GC, bs1, br1, g1)

        def body(i, carry):
            j0 = 2 * i
            drain_g(_GC, bs0, br0, g0)

            @pl.when(i > 0)
            def _():
                drain_w(_GC, ob0, w0)

            add(_GC, bs0, br0, ob0)

            @pl.when(j0 + 2 < _NGF)
            def _():
                issue(j0 + 2, _GC, bs0, br0, g0)

            @pl.when(j0 + 2 == _NGF)
            def _():
                issue(j0 + 2, _TAIL, bs0, br0, g0)

            issue_w(j0, _GC, ob0, w0)

            drain_g(_GC, bs1, br1, g1)

            @pl.when(i > 0)
            def _():
                drain_w(_GC, ob1, w1)

            add(_GC, bs1, br1, ob1)

            @pl.when(j0 + 3 < _NGF)
            def _():
                issue(j0 + 3, _GC, bs1, br1, g1)

            @pl.when(j0 + 3 == _NGF)
            def _():
                issue(j0 + 3, _TAIL, bs1, br1, g1)

            issue_w(j0 + 1, _GC, ob1, w1)
            return carry

        lax.fori_loop(0, _NGF // 2, body, 0)
        drain_g(_TAIL, bs0, br0, g0)
        drain_w(_GC, ob0, w0)
        add(_TAIL, bs0, br0, ob0)
        issue_w(_NGF, _TAIL, ob0, w0)
        drain_w(_TAIL, ob0, w0)
        drain_w(_GC, ob1, w1)

    return k


@functools.cache
def _scatter_add_kernel():
    @functools.partial(
        pl.kernel,
        out_type=jax.ShapeDtypeStruct((_NC, _NPAD, _LAT), jnp.float32),
        mesh=_sc_mesh(),
        scratch_types=[
            pltpu.VMEM((_NCH, _C), jnp.int32),
            pltpu.VMEM((_C, _LAT), jnp.float32),
            pltpu.VMEM((_C, _LAT), jnp.float32),
            pltpu.VMEM((_ZR, _LAT), jnp.float32),
            pltpu.VMEM_SHARED((_NPAD, _LAT), jnp.float32),
            pltpu.SemaphoreType.DMA,
            pltpu.SemaphoreType.DMA,
        ],
    )
    def k(ne_hbm, rcv_hbm, out_hbm, idx, b0, b1, zbuf, acc, r0sem, r1sem):
        c = lax.axis_index("c")
        s = lax.axis_index("s")
        wid = c * _NS + s
        base = wid * _EPW
        zero16 = jnp.zeros((16,), jnp.float32)

        pltpu.sync_copy(rcv_hbm.at[wid], idx)

        def zrow(r, carry):
            for j in range(_LAT // 16):
                zbuf[r, pl.ds(j * 16, 16)] = zero16
            return carry

        lax.fori_loop(0, _ZR, zrow, 0)
        my_r0 = s * _RPT

        def zchunk(i, carry):
            pltpu.sync_copy(zbuf, acc.at[pl.ds(my_r0 + i * _ZR, _ZR)])
            return carry

        lax.fori_loop(0, _RPT // _ZR, zchunk, 0)
        plsc.subcore_barrier()

        def issue(j, b, sem):
            row0 = pl.multiple_of(base + j * _C, 8)
            pltpu.async_copy(ne_hbm.at[pl.ds(row0, _C)], b, sem)

        def drain(b, sem):
            pltpu.make_async_copy(ne_hbm.at[pl.ds(base, _C)], b, sem).wait()

        issue(0, b0, r0sem)

        def body(i, carry):
            j1 = 2 * i + 1
            issue(j1, b1, r1sem)
            drain(b0, r0sem)
            pltpu.sync_copy(b0, acc.at[idx.at[2 * i]], add=True)
            issue(j1 + 1, b0, r0sem)
            drain(b1, r1sem)
            pltpu.sync_copy(b1, acc.at[idx.at[j1]], add=True)
            return carry

        lax.fori_loop(0, (_NCH - 1) // 2, body, 0)
        drain(b0, r0sem)
        pltpu.sync_copy(b0, acc.at[idx.at[_NCH - 1]], add=True)
        plsc.subcore_barrier()

        def wchunk(i, carry):
            r0 = my_r0 + i * _ZR
            pltpu.sync_copy(acc.at[pl.ds(r0, _ZR)], out_hbm.at[c, pl.ds(r0, _ZR)])
            return carry

        lax.fori_loop(0, _RPT // _ZR, wchunk, 0)

    return k


def _sc_gather_add(sw, rw, snd3, rcv3):
    return _gather_add_kernel()(sw, rw, snd3, rcv3)


def _sc_scatter_partials(ne, rcv3):
    return _scatter_add_kernel()(ne, rcv3)


def _r(b):
    return b.reshape(1, -1)


def kernel(node_features, edge_features, senders, receivers, params):
    p = params
    (nw1, nb1), (nw2, nb2), (nw3, nb3) = p['node_enc']['mlp']
    ng, nb = p['node_enc']['ln']
    (ew1, eb1), (ew2, eb2), (ew3, eb3) = p['edge_enc']['mlp']
    eg, eb = p['edge_enc']['ln']
    (d1, f1), (d2, f2), (d3, f3) = p['dec']

    blocks = []
    for bp in p['blocks']:
        (w1e, b1e), (w2e, b2e), (w3e, b3e) = bp['edge']['mlp']
        (w1n, b1n), (w2n, b2n), (w3n, b3n) = bp['node']['mlp']
        blocks.append(dict(
            ws=w1e[:_LAT], wr=w1e[_LAT:2 * _LAT], we=w1e[2 * _LAT:],
            b1e=_r(b1e), w2e=w2e, b2e=_r(b2e), w3e=w3e, b3e=_r(b3e),
            ge=_r(bp['edge']['ln'][0]), be=_r(bp['edge']['ln'][1]),
            wn=w1n[:_LAT], wa=w1n[_LAT:], b1n=_r(b1n),
            w2n=w2n, b2n=_r(b2n), w3n=w3n, b3n=_r(b3n),
            gn=_r(bp['node']['ln'][0]), bn=_r(bp['node']['ln'][1]),
        ))

    d3p = jnp.zeros((_LAT, _LAT), jnp.float32).at[:, :_OUT].set(d3)
    f3p = jnp.zeros((1, _LAT), jnp.float32).at[:, :_OUT].set(_r(f3))

    node_lat, sw, rw = _tc_call(
        _node_enc_body, [node_features],
        [nw1, _r(nb1), nw2, _r(nb2), nw3, _r(nb3), _r(ng), _r(nb),
         blocks[0]['ws'], blocks[0]['wr'], blocks[0]['b1e']],
        [(_N, _LAT)] * 3, _RN, _N)

    ew_args = [ew1, _r(eb1), ew2, _r(eb2), ew3, _r(eb3), _r(eg), _r(eb)]

    def edge_enc(eft_half):
        return pl.pallas_call(
            _edge_enc_body,
            grid=(_EH // _RE,),
            in_specs=([pl.BlockSpec((4, _RE), lambda i: (0, i))]
                      + [_full_spec(a) for a in ew_args]),
            out_specs=_row_spec(_RE, _LAT),
            out_shape=jax.ShapeDtypeStruct((_EH, _LAT), jnp.float32),
            compiler_params=pltpu.CompilerParams(
                dimension_semantics=("parallel",)),
        )(eft_half, *ew_args)

    elat = [edge_enc(edge_features[:_EH].T), edge_enc(edge_features[_EH:].T)]
    idx_s = [senders[:_EH].reshape(_NW, -1),
             senders[_EH:].reshape(_NW, -1)]
    idx_r = [receivers[:_EH].reshape(_NW, -1),
             receivers[_EH:].reshape(_NW, -1)]
    idx_r3 = [receivers[:_EH].reshape(_NW, -1, _C),
              receivers[_EH:].reshape(_NW, -1, _C)]

    for s, bl in enumerate(blocks):
        ew = [bl['we'], bl['w2e'], bl['b2e'], bl['w3e'], bl['b3e'],
              bl['ge'], bl['be']]
        g = [_sc_gather_add(sw, rw, idx_s[h], idx_r[h]) for h in (0, 1)]
        ne = [None, None]
        parts = [None, None]
        for h in (0, 1):
            if s < len(blocks) - 1:
                ne[h], elat[h] = _tc_call(
                    _edge_step_body, [g[h], elat[h]], ew,
                    [(_EH, _LAT)] * 2, _RE, _EH)
            else:
                ne[h] = _tc_call(
                    _edge_last_body, [g[h], elat[h]], ew,
                    [(_EH, _LAT)], _RE, _EH)
            parts[h] = _sc_scatter_partials(ne[h], idx_r3[h])
        if s < len(blocks) - 1:
            nxt = blocks[s + 1]
            node_lat, sw, rw = _tc_call(
                _node_step_body, [node_lat, parts[0], parts[1]],
                [bl['wn'], bl['wa'], bl['b1n'], bl['w2n'], bl['b2n'],
                 bl['w3n'], bl['b3n'], bl['gn'], bl['bn'],
                 nxt['ws'], nxt['wr'], nxt['b1e']],
                [(_N, _LAT)] * 3, _RN, _N)
        else:
            dec = _tc_call(
                _node_final_body, [node_lat, parts[0], parts[1]],
                [bl['wn'], bl['wa'], bl['b1n'], bl['w2n'], bl['b2n'],
                 bl['w3n'], bl['b3n'], bl['gn'], bl['bn'],
                 d1, _r(f1), d2, _r(f2), d3p, f3p],
                [(_N, _LAT)], _RN, _N)
    return dec[:, :_OUT]


# speedup vs baseline: 4.3686x; 1.0296x over previous
import functools

import jax
import jax.numpy as jnp
from jax import lax
from jax.experimental import pallas as pl
from jax.experimental.pallas import tpu as pltpu
from jax.experimental.pallas import tpu_sc as plsc

_N = 10000
_E = 320000
_LAT = 128
_OUT = 3
_EPS = 1e-5

_EH = _E // 2
_RE = 1280
_RN = 2000
_NC, _NS = 2, 16
_NW = _NC * _NS
_EPW = _EH // _NW
_C = 40
_NPAD = 10240
_RPT = _NPAD // _NS
_ZR = 128


def _ln(y, g, b):
    mu = jnp.mean(y, axis=-1, keepdims=True)
    yc = y - mu
    var = jnp.mean(yc * yc, axis=-1, keepdims=True)
    return yc / jnp.sqrt(var + _EPS) * g + b


def _dot(x, w):
    return jnp.dot(x, w, preferred_element_type=jnp.float32)


def _node_enc_body(x, w1, b1, w2, b2, w3, b3, g, bb, ws, wr, b1e,
                   o_lat, o_sw, o_rw):
    h = jnp.maximum(_dot(x[...], w1[...]) + b1[...], 0.0)
    h = jnp.maximum(_dot(h, w2[...]) + b2[...], 0.0)
    y = _ln(_dot(h, w3[...]) + b3[...], g[...], bb[...])
    o_lat[...] = y
    o_sw[...] = _dot(y, ws[...]) + b1e[...]
    o_rw[...] = _dot(y, wr[...])


def _edge_enc_body(xt, w1, b1, w2, b2, w3, b3, g, bb, o_lat):
    h = lax.dot_general(xt[...], w1[...], (((0,), (0,)), ((), ())),
                        preferred_element_type=jnp.float32)
    h = jnp.maximum(h + b1[...], 0.0)
    h = jnp.maximum(_dot(h, w2[...]) + b2[...], 0.0)
    o_lat[...] = _ln(_dot(h, w3[...]) + b3[...], g[...], bb[...])


def _edge_step_body(g_in, elat, we, w2, b2, w3, b3, lg, lb, o_ne, o_eout):
    e = elat[...]
    h = jnp.maximum(g_in[...] + _dot(e, we[...]), 0.0)
    h = jnp.maximum(_dot(h, w2[...]) + b2[...], 0.0)
    ne = _ln(_dot(h, w3[...]) + b3[...], lg[...], lb[...])
    o_ne[...] = ne
    o_eout[...] = e + ne


def _edge_last_body(g_in, elat, we, w2, b2, w3, b3, lg, lb, o_ne):
    h = jnp.maximum(g_in[...] + _dot(elat[...], we[...]), 0.0)
    h = jnp.maximum(_dot(h, w2[...]) + b2[...], 0.0)
    o_ne[...] = _ln(_dot(h, w3[...]) + b3[...], lg[...], lb[...])


def _node_step_body(nlat, pa, pb, wn, wa, b1, w2, b2, w3, b3, g, bb,
                    ws, wr, b1e, o_lat, o_sw, o_rw):
    n = nlat[...]
    agg = (pa[0] + pa[1]) + (pb[0] + pb[1])
    h = jnp.maximum(_dot(n, wn[...]) + _dot(agg, wa[...]) + b1[...], 0.0)
    h = jnp.maximum(_dot(h, w2[...]) + b2[...], 0.0)
    y = _ln(_dot(h, w3[...]) + b3[...], g[...], bb[...])
    no = n + y
    o_lat[...] = no
    o_sw[...] = _dot(no, ws[...]) + b1e[...]
    o_rw[...] = _dot(no, wr[...])


def _node_final_body(nlat, pa, pb, wn, wa, b1, w2, b2, w3, b3, g, bb,
                     d1, e1, d2, e2, d3, e3, o_dec):
    n = nlat[...]
    agg = (pa[0] + pa[1]) + (pb[0] + pb[1])
    h = jnp.maximum(_dot(n, wn[...]) + _dot(agg, wa[...]) + b1[...], 0.0)
    h = jnp.maximum(_dot(h, w2[...]) + b2[...], 0.0)
    y = _ln(_dot(h, w3[...]) + b3[...], g[...], bb[...])
    no = n + y
    h = jnp.maximum(_dot(no, d1[...]) + e1[...], 0.0)
    h = jnp.maximum(_dot(h, d2[...]) + e2[...], 0.0)
    o_dec[...] = _dot(h, d3[...]) + e3[...]


def _full_spec(a):
    nd = a.ndim
    return pl.BlockSpec(a.shape, lambda i, _nd=nd: (0,) * _nd)


def _row_spec(rows, cols):
    return pl.BlockSpec((rows, cols), lambda i: (i, 0))


def _tc_call(body, row_args, weight_args, out_shapes, rows, total_rows):
    grid = (total_rows // rows,)
    in_specs = [
        (pl.BlockSpec((_NC, rows, a.shape[-1]), lambda i: (0, i, 0))
         if a.ndim == 3 else _row_spec(rows, a.shape[-1]))
        for a in row_args
    ]
    in_specs += [_full_spec(a) for a in weight_args]
    out_specs = [_row_spec(rows, s[-1]) for s in out_shapes]
    outs = pl.pallas_call(
        body,
        grid=grid,
        in_specs=in_specs,
        out_specs=out_specs if len(out_specs) > 1 else out_specs[0],
        out_shape=([jax.ShapeDtypeStruct(s, jnp.float32) for s in out_shapes]
                   if len(out_shapes) > 1
                   else jax.ShapeDtypeStruct(out_shapes[0], jnp.float32)),
        compiler_params=pltpu.CompilerParams(
            dimension_semantics=("parallel",)),
    )(*row_args, *weight_args)
    return outs


@functools.cache
def _sc_mesh():
    return plsc.VectorSubcoreMesh(core_axis_name="c", subcore_axis_name="s",
                                  num_cores=_NC, num_subcores=_NS)


_NCH = _EPW // _C


_GC = 80
_NGF = _EPW // _GC
_TAIL = _EPW - _NGF * _GC


@functools.cache
def _gather_add_kernel():
    @functools.partial(
        pl.kernel,
        out_type=jax.ShapeDtypeStruct((_EH, _LAT), jnp.float32),
        mesh=_sc_mesh(),
        scratch_types=[
            pltpu.VMEM((_EPW,), jnp.int32),
            pltpu.VMEM((_EPW,), jnp.int32),
            pltpu.VMEM((_GC, _LAT), jnp.float32),
            pltpu.VMEM((_GC, _LAT), jnp.float32),
            pltpu.VMEM((_GC, _LAT), jnp.float32),
            pltpu.VMEM((_GC, _LAT), jnp.float32),
            pltpu.VMEM((_GC, _LAT), jnp.float32),
            pltpu.VMEM((_GC, _LAT), jnp.float32),
            pltpu.SemaphoreType.DMA,
            pltpu.SemaphoreType.DMA,
            pltpu.SemaphoreType.DMA,
            pltpu.SemaphoreType.DMA,
        ],
    )
    def k(sw_hbm, rw_hbm, snd_hbm, rcv_hbm, out_hbm,
          idx_s, idx_r, bs0, br0, bs1, br1, ob0, ob1, g0, g1, w0, w1):
        c = lax.axis_index("c")
        s = lax.axis_index("s")
        wid = c * _NS + s
        base = wid * _EPW

        pltpu.sync_copy(snd_hbm.at[wid], idx_s)
        pltpu.sync_copy(rcv_hbm.at[wid], idx_r)

        def issue(j, n, bs, br, sem):
            off = pl.multiple_of(j * _GC, 8)
            pltpu.async_copy(sw_hbm.at[idx_s.at[pl.ds(off, n)]],
                             bs.at[pl.ds(0, n)], sem)
            pltpu.async_copy(rw_hbm.at[idx_r.at[pl.ds(off, n)]],
                             br.at[pl.ds(0, n)], sem)

        def drain_g(n, bs, br, sem):
            pltpu.make_async_copy(sw_hbm.at[idx_s.at[pl.ds(0, n)]],
                                  bs.at[pl.ds(0, n)], sem).wait()
            pltpu.make_async_copy(rw_hbm.at[idx_r.at[pl.ds(0, n)]],
                                  br.at[pl.ds(0, n)], sem).wait()

        def add(n, bs, br, ob):
            def addrow(r, carry2):
                for q in range(_LAT // 16):
                    sl = pl.ds(q * 16, 16)
                    ob[r, sl] = bs[r, sl] + br[r, sl]
                return carry2

            lax.fori_loop(0, n, addrow, 0, unroll=2)

        def issue_w(j, n, ob, wsem):
            row0 = pl.multiple_of(base + j * _GC, 8)
            pltpu.async_copy(ob.at[pl.ds(0, n)],
                             out_hbm.at[pl.ds(row0, n)], wsem)

        def drain_w(n, ob, wsem):
            pltpu.make_async_copy(ob.at[pl.ds(0, n)],
                                  out_hbm.at[pl.ds(base, n)], wsem).wait()

        issue(0, _GC, bs0, br0, g0)
        issue(1, _GC, bs1, br1, g1)

        def body(i, carry):
            j0 = 2 * i
            drain_g(_GC, bs0, br0, g0)

            @pl.when(i > 0)
            def _():
                drain_w(_GC, ob0, w0)

            add(_GC, bs0, br0, ob0)

            @pl.when(j0 + 2 < _NGF)
            def _():
                issue(j0 + 2, _GC, bs0, br0, g0)

            @pl.when(j0 + 2 == _NGF)
            def _():
                issue(j0 + 2, _TAIL, bs0, br0, g0)

            issue_w(j0, _GC, ob0, w0)

            drain_g(_GC, bs1, br1, g1)

            @pl.when(i > 0)
            def _():
                drain_w(_GC, ob1, w1)

            add(_GC, bs1, br1, ob1)

            @pl.when(j0 + 3 < _NGF)
            def _():
                issue(j0 + 3, _GC, bs1, br1, g1)

            @pl.when(j0 + 3 == _NGF)
            def _():
                issue(j0 + 3, _TAIL, bs1, br1, g1)

            issue_w(j0 + 1, _GC, ob1, w1)
            return carry

        lax.fori_loop(0, _NGF // 2, body, 0)
        drain_g(_TAIL, bs0, br0, g0)
        drain_w(_GC, ob0, w0)
        add(_TAIL, bs0, br0, ob0)
        issue_w(_NGF, _TAIL, ob0, w0)
        drain_w(_TAIL, ob0, w0)
        drain_w(_GC, ob1, w1)

    return k


@functools.cache
def _scatter_add_kernel():
    @functools.partial(
        pl.kernel,
        out_type=jax.ShapeDtypeStruct((_NC, _NPAD, _LAT), jnp.float32),
        mesh=_sc_mesh(),
        scratch_types=[
            pltpu.VMEM((_NCH, _C), jnp.int32),
            pltpu.VMEM((_C, _LAT), jnp.float32),
            pltpu.VMEM((_C, _LAT), jnp.float32),
            pltpu.VMEM((_C, _LAT), jnp.float32),
            pltpu.VMEM((_C, _LAT), jnp.float32),
            pltpu.VMEM((_C, _LAT), jnp.float32),
            pltpu.VMEM_SHARED((_NPAD, _LAT), jnp.float32),
            pltpu.SemaphoreType.DMA,
            pltpu.SemaphoreType.DMA,
            pltpu.SemaphoreType.DMA,
            pltpu.SemaphoreType.DMA,
            pltpu.SemaphoreType.DMA,
            pltpu.SemaphoreType.DMA,
            pltpu.SemaphoreType.DMA,
            pltpu.SemaphoreType.DMA,
            pltpu.SemaphoreType.DMA,
            pltpu.SemaphoreType.DMA,
        ],
    )
    def k(ne_hbm, rcv_hbm, out_hbm, idx, b0, b1, b2, b3, b4, acc,
          r0, r1, r2, r3, r4, s0, s1, s2, s3, s4):
        c = lax.axis_index("c")
        s = lax.axis_index("s")
        wid = c * _NS + s
        base = wid * _EPW
        zero16 = jnp.zeros((16,), jnp.float32)
        bufs = (b0, b1, b2, b3, b4)
        rsems = (r0, r1, r2, r3, r4)
        ssems = (s0, s1, s2, s3, s4)

        pltpu.sync_copy(rcv_hbm.at[wid], idx)

        def zrow(r, carry):
            for j in range(_LAT // 16):
                b0[r, pl.ds(j * 16, 16)] = zero16
            return carry

        lax.fori_loop(0, _C, zrow, 0)
        my_r0 = s * _RPT

        def zchunk(i, carry):
            pltpu.sync_copy(b0, acc.at[pl.ds(my_r0 + i * _C, _C)])
            return carry

        lax.fori_loop(0, _RPT // _C, zchunk, 0)
        plsc.subcore_barrier()

        def issue(j, b, sem):
            row0 = pl.multiple_of(base + j * _C, 8)
            pltpu.async_copy(ne_hbm.at[pl.ds(row0, _C)], b, sem)

        def drain_r(b, sem):
            pltpu.make_async_copy(ne_hbm.at[pl.ds(base, _C)], b, sem).wait()

        def drain_s(b, sem):
            pltpu.make_async_copy(b, acc.at[idx.at[0]], sem).wait()

        for p in range(3):
            issue(p, bufs[p], rsems[p])
        nb = _NCH // 5

        def body(i, carry):
            for p in range(5):
                j = 5 * i + p
                q = (p + 3) % 5
                drain_r(bufs[p], rsems[p])
                pltpu.async_copy(bufs[p], acc.at[idx.at[j]], ssems[p],
                                 add=True)
                if p < 2:
                    @pl.when(i > 0)
                    def _(q=q):
                        drain_s(bufs[q], ssems[q])

                    issue(j + 3, bufs[q], rsems[q])
                else:
                    drain_s(bufs[q], ssems[q])

                    @pl.when(i < nb - 1)
                    def _(q=q, j=j):
                        issue(j + 3, bufs[q], rsems[q])

            return carry

        lax.fori_loop(0, nb, body, 0)
        drain_s(bufs[3], ssems[3])
        drain_s(bufs[4], ssems[4])
        plsc.subcore_barrier()

        def wchunk(i, carry):
            r0 = my_r0 + i * _ZR
            pltpu.sync_copy(acc.at[pl.ds(r0, _ZR)], out_hbm.at[c, pl.ds(r0, _ZR)])
            return carry

        lax.fori_loop(0, _RPT // _ZR, wchunk, 0)

    return k


def _sc_gather_add(sw, rw, snd3, rcv3):
    return _gather_add_kernel()(sw, rw, snd3, rcv3)


def _sc_scatter_partials(ne, rcv3):
    return _scatter_add_kernel()(ne, rcv3)


def _r(b):
    return b.reshape(1, -1)


def kernel(node_features, edge_features, senders, receivers, params):
    p = params
    (nw1, nb1), (nw2, nb2), (nw3, nb3) = p['node_enc']['mlp']
    ng, nb = p['node_enc']['ln']
    (ew1, eb1), (ew2, eb2), (ew3, eb3) = p['edge_enc']['mlp']
    eg, eb = p['edge_enc']['ln']
    (d1, f1), (d2, f2), (d3, f3) = p['dec']

    blocks = []
    for bp in p['blocks']:
        (w1e, b1e), (w2e, b2e), (w3e, b3e) = bp['edge']['mlp']
        (w1n, b1n), (w2n, b2n), (w3n, b3n) = bp['node']['mlp']
        blocks.append(dict(
            ws=w1e[:_LAT], wr=w1e[_LAT:2 * _LAT], we=w1e[2 * _LAT:],
            b1e=_r(b1e), w2e=w2e, b2e=_r(b2e), w3e=w3e, b3e=_r(b3e),
            ge=_r(bp['edge']['ln'][0]), be=_r(bp['edge']['ln'][1]),
            wn=w1n[:_LAT], wa=w1n[_LAT:], b1n=_r(b1n),
            w2n=w2n, b2n=_r(b2n), w3n=w3n, b3n=_r(b3n),
            gn=_r(bp['node']['ln'][0]), bn=_r(bp['node']['ln'][1]),
        ))

    d3p = jnp.zeros((_LAT, _LAT), jnp.float32).at[:, :_OUT].set(d3)
    f3p = jnp.zeros((1, _LAT), jnp.float32).at[:, :_OUT].set(_r(f3))

    node_lat, sw, rw = _tc_call(
        _node_enc_body, [node_features],
        [nw1, _r(nb1), nw2, _r(nb2), nw3, _r(nb3), _r(ng), _r(nb),
         blocks[0]['ws'], blocks[0]['wr'], blocks[0]['b1e']],
        [(_N, _LAT)] * 3, _RN, _N)

    ew_args = [ew1, _r(eb1), ew2, _r(eb2), ew3, _r(eb3), _r(eg), _r(eb)]

    def edge_enc(eft_half):
        return pl.pallas_call(
            _edge_enc_body,
            grid=(_EH // _RE,),
            in_specs=([pl.BlockSpec((4, _RE), lambda i: (0, i))]
                      + [_full_spec(a) for a in ew_args]),
            out_specs=_row_spec(_RE, _LAT),
            out_shape=jax.ShapeDtypeStruct((_EH, _LAT), jnp.float32),
            compiler_params=pltpu.CompilerParams(
                dimension_semantics=("parallel",)),
        )(eft_half, *ew_args)

    elat = [edge_enc(edge_features[:_EH].T), edge_enc(edge_features[_EH:].T)]
    idx_s = [senders[:_EH].reshape(_NW, -1),
             senders[_EH:].reshape(_NW, -1)]
    idx_r = [receivers[:_EH].reshape(_NW, -1),
             receivers[_EH:].reshape(_NW, -1)]
    idx_r3 = [receivers[:_EH].reshape(_NW, -1, _C),
              receivers[_EH:].reshape(_NW, -1, _C)]

    for s, bl in enumerate(blocks):
        ew = [bl['we'], bl['w2e'], bl['b2e'], bl['w3e'], bl['b3e'],
              bl['ge'], bl['be']]
        g = [_sc_gather_add(sw, rw, idx_s[h], idx_r[h]) for h in (0, 1)]
        ne = [None, None]
        parts = [None, None]
        for h in (0, 1):
            if s < len(blocks) - 1:
                ne[h], elat[h] = _tc_call(
                    _edge_step_body, [g[h], elat[h]], ew,
                    [(_EH, _LAT)] * 2, _RE, _EH)
            else:
                ne[h] = _tc_call(
                    _edge_last_body, [g[h], elat[h]], ew,
                    [(_EH, _LAT)], _RE, _EH)
            parts[h] = _sc_scatter_partials(ne[h], idx_r3[h])
        if s < len(blocks) - 1:
            nxt = blocks[s + 1]
            node_lat, sw, rw = _tc_call(
                _node_step_body, [node_lat, parts[0], parts[1]],
                [bl['wn'], bl['wa'], bl['b1n'], bl['w2n'], bl['b2n'],
                 bl['w3n'], bl['b3n'], bl['gn'], bl['bn'],
                 nxt['ws'], nxt['wr'], nxt['b1e']],
                [(_N, _LAT)] * 3, _RN, _N)
        else:
            dec = _tc_call(
                _node_final_body, [node_lat, parts[0], parts[1]],
                [bl['wn'], bl['wa'], bl['b1n'], bl['w2n'], bl['b2n'],
                 bl['w3n'], bl['b3n'], bl['gn'], bl['bn'],
                 d1, _r(f1), d2, _r(f2), d3p, f3p],
                [(_N, _LAT)], _RN, _N)
    return dec[:, :_OUT]
